# Initial kernel scaffold; baseline (speedup 1.0000x reference)
import jax
import jax.numpy as jnp
from jax.experimental import pallas as pl


def kernel(x, edge_index, W1, a_src1, a_dst1, b1, W2, a_src2, a_dst2, b2, W_fc, b_fc):
    raise NotImplementedError("write your pallas kernel here")


# speedup vs baseline: 20.8673x; 20.8673x over previous
import jax
import jax.numpy as jnp
from jax import lax
from jax.experimental import pallas as pl
from jax.experimental.pallas import tpu as pltpu
from jax.experimental.pallas import tpu_sc as plsc

_N = 10000
_E = 320000
_HEADS = 4

_K = 128
_NTILES = 16
_NCORES = 2

_EP = _NCORES * _NTILES * _K * ((_E + _NCORES * _NTILES * _K - 1) // (_NCORES * _NTILES * _K))
_ET1 = _EP // _NTILES
_NC1 = _ET1 // _K
_ET2 = _EP // (_NCORES * _NTILES)
_NC2 = _ET2 // _K

_NP = 10240
_ROWS_PER_TILE = _NP // _NTILES


def _iota16():
    return lax.iota(jnp.int32, 16)


def _tc1_body(x_ref, w1_ref, as_ref, ad_ref, hcat_ref, asrc_ref, adst_ref,
              ctbl_ref, root_ref):
    x = x_ref[...]
    h = jnp.dot(x, w1_ref[...], preferred_element_type=jnp.float32)
    for hd in range(4):
        hcat_ref[hd * _N:(hd + 1) * _N, :] = h[:, hd * 64:(hd + 1) * 64]
    asrc = jnp.dot(h, as_ref[...], preferred_element_type=jnp.float32)
    adst = jnp.dot(h, ad_ref[...], preferred_element_type=jnp.float32)
    asrc_ref[...] = asrc
    adst_ref[...] = adst
    c = jnp.maximum(jnp.max(asrc, axis=0) + jnp.max(adst, axis=0), 0.0)
    ctbl_ref[...] = jnp.broadcast_to(c[:, None], (4, 16))
    mask = x[:, 0:1] == 0.0
    ids = lax.broadcasted_iota(jnp.int32, (_N, 1), 0)
    rid = jnp.min(jnp.where(mask, ids, _N))
    root_ref[...] = jnp.broadcast_to(jnp.where(rid == _N, 0, rid), (1, 1))


def _tc1(x, w1, a_s, a_d):
    return pl.pallas_call(
        _tc1_body,
        out_shape=[
            jax.ShapeDtypeStruct((4 * _N, 64), jnp.float32),
            jax.ShapeDtypeStruct((_N, 4), jnp.float32),
            jax.ShapeDtypeStruct((_N, 4), jnp.float32),
            jax.ShapeDtypeStruct((4, 16), jnp.float32),
            jax.ShapeDtypeStruct((1, 1), jnp.int32),
        ],
    )(x, w1, a_s, a_d)


_B2 = 2000


def _tc2_body(a0l_ref, a1l_ref, a0h_ref, a1h_ref, adl_ref, adh_ref,
              b1_ref, w2_ref, as2_ref, ad2_ref,
              h2_ref, asrc2_ref, adst2_ref):
    den = adl_ref[:, 0:4] + adh_ref[:, 0:4] + 1e-16
    num = jnp.concatenate([a0l_ref[...], a1l_ref[...],
                           a0h_ref[...], a1h_ref[...]], axis=1)
    scale = jnp.concatenate(
        [jnp.broadcast_to(1.0 / den[:, hd:hd + 1], (_B2, 64)) for hd in range(4)],
        axis=1)
    h1 = jnp.maximum(num * scale + b1_ref[...][None, :], 0.0)
    h2 = jnp.dot(h1, w2_ref[...], preferred_element_type=jnp.float32)
    h2_ref[...] = h2
    asrc2_ref[...] = jnp.sum(h2 * as2_ref[...], axis=1, keepdims=True)
    adst2_ref[...] = jnp.sum(h2 * ad2_ref[...], axis=1, keepdims=True)


def _tc2(a0l, a1l, a0h, a1h, adl, adh, b1, w2, a_s2, a_d2):
    nblk = _N // _B2
    row = lambda i: (i, 0)
    full2 = lambda i: (0, 0)
    return pl.pallas_call(
        _tc2_body,
        grid=(nblk,),
        in_specs=[
            pl.BlockSpec((_B2, 64), row), pl.BlockSpec((_B2, 64), row),
            pl.BlockSpec((_B2, 64), row), pl.BlockSpec((_B2, 64), row),
            pl.BlockSpec((_B2, 16), row), pl.BlockSpec((_B2, 16), row),
            pl.BlockSpec((256,), lambda i: (0,)),
            pl.BlockSpec((256, 64), full2),
            pl.BlockSpec((1, 64), full2), pl.BlockSpec((1, 64), full2),
        ],
        out_specs=[
            pl.BlockSpec((_B2, 64), row),
            pl.BlockSpec((_B2, 1), row),
            pl.BlockSpec((_B2, 1), row),
        ],
        out_shape=[
            jax.ShapeDtypeStruct((_N, 64), jnp.float32),
            jax.ShapeDtypeStruct((_N, 1), jnp.float32),
            jax.ShapeDtypeStruct((_N, 1), jnp.float32),
        ],
    )(a0l, a1l, a0h, a1h, adl, adh, b1, w2, a_s2, a_d2)


def _tcc2_body(asrc2_ref, adst2_ref, ctbl2_ref):
    c2 = jnp.maximum(jnp.max(asrc2_ref[...]) + jnp.max(adst2_ref[...]), 0.0)
    ctbl2_ref[...] = jnp.full((1, 16), 1.0) * c2


def _tcc2(asrc2, adst2):
    return pl.pallas_call(
        _tcc2_body,
        out_shape=jax.ShapeDtypeStruct((1, 16), jnp.float32),
    )(asrc2, adst2)


def _tc3_body(m2l_ref, m2h_ref, d2l_ref, b2_ref, wfc_ref, bfc_ref, out_ref):
    num = jnp.concatenate([m2l_ref[...], m2h_ref[...]], axis=1)
    den = d2l_ref[:, 0:1] + 1e-16
    h2 = jnp.maximum(num / den + b2_ref[...][None, :], 0.0)
    out_ref[...] = (jnp.dot(h2, wfc_ref[...], preferred_element_type=jnp.float32)
                    + bfc_ref[...][None, :])


def _tc3(m2l, m2h, d2l, b2, wfc, bfc):
    return pl.pallas_call(
        _tc3_body,
        out_shape=jax.ShapeDtypeStruct((_N, 64), jnp.float32),
    )(m2l, m2h, d2l, b2, wfc, bfc)


_SC_PARAMS = pltpu.CompilerParams(needs_layout_passes=False,
                                  use_tc_tiling_on_sc=False)
_MESH = dict(core_axis_name="c", subcore_axis_name="s")


def _zero_acc(rows_v, pad_v, accm_s, accd_s, s, nvec):
    def _zrow(i, _):
        if rows_v is not None:
            for v in range(nvec):
                rows_v[i, pl.ds(v * 16, 16)] = jnp.zeros((16,), jnp.float32)
        if pad_v is not None:
            pad_v[i, :] = jnp.zeros((16,), jnp.float32)
        return 0
    lax.fori_loop(0, _K, _zrow, 0)
    for r in range(0, _ROWS_PER_TILE, _K):
        if accm_s is not None:
            pltpu.sync_copy(rows_v, accm_s.at[pl.ds(s * _ROWS_PER_TILE + r, _K), :])
        if accd_s is not None:
            pltpu.sync_copy(pad_v, accd_s.at[pl.ds(s * _ROWS_PER_TILE + r, _K), :])
    plsc.subcore_barrier()


def _write_acc(acc_s, buf_v, out_hbm, s, cP):
    for r in range(0, _ROWS_PER_TILE, _K):
        row0 = s * _ROWS_PER_TILE + r
        pltpu.sync_copy(acc_s.at[pl.ds(row0, _K), :], buf_v)
        pltpu.sync_copy(buf_v, out_hbm.at[pl.ds(cP + row0, _K), :])


def _edge_w(asrc_v, adst_v, ctbl_v, src_b, dst_b, wtmp_v, base, heads, nh):
    for g in range(8):
        sv = src_b[pl.ds(g * 16, 16)] * nh
        dv = dst_b[pl.ds(g * 16, 16)] * nh
        eid = base + g * 16 + _iota16()
        valid = eid < _E
        for hd in heads:
            e = (plsc.load_gather(asrc_v, [sv + hd])
                 + plsc.load_gather(adst_v, [dv + hd]))
            e = jnp.maximum(e, 0.2 * e) - ctbl_v[pl.ds(hd * 16, 16)]
            w = jnp.where(valid, jnp.exp(e), 0.0)
            wtmp_v[pl.ds(hd * _K + g * 16, 16)] = w


def _sc_den1_body(asrc, adst, ctbl, srcp, dstp, accd_out,
                  asrc_v, adst_v, ctbl_v, src_b, dst_b, pad_v, wtmp_v, accd_s,
                  sem):
    c = lax.axis_index("c")
    s = lax.axis_index("s")
    cP = c * _NP
    pltpu.sync_copy(asrc, asrc_v)
    pltpu.sync_copy(adst, adst_v)
    pltpu.sync_copy(ctbl, ctbl_v)
    _zero_acc(None, pad_v, None, accd_s, s, 0)

    tile_base = (c * _NTILES + s) * _ET2
    lanes = _iota16()

    def _chunk(j, _):
        base = tile_base + j * _K
        pltpu.sync_copy(srcp.at[pl.ds(base, _K)], src_b)
        pltpu.sync_copy(dstp.at[pl.ds(base, _K)], dst_b)
        _edge_w(asrc_v, adst_v, ctbl_v, src_b, dst_b, wtmp_v, base,
                (0, 1, 2, 3), 4)

        def _edge(k, _):
            kv = jnp.full((16,), k, jnp.int32)
            padv = plsc.load_gather(wtmp_v, [(lanes & 3) * _K + kv])
            pad_v[k, :] = jnp.where(lanes < 4, padv, 0.0)
            return 0
        lax.fori_loop(0, _K, _edge, 0)

        pltpu.sync_copy(pad_v, accd_s.at[dst_b], add=True)
        return 0

    lax.fori_loop(0, _NC2, _chunk, 0)
    plsc.subcore_barrier()
    _write_acc(accd_s, pad_v, accd_out, s, cP)


def _sc_den1(asrc, adst, ctbl, srcp, dstp):
    f = pl.kernel(
        _sc_den1_body, mesh=plsc.VectorSubcoreMesh(**_MESH),
        compiler_params=_SC_PARAMS,
        out_type=[jax.ShapeDtypeStruct((2 * _NP, 16), jnp.float32)],
        scratch_types=[
            pltpu.VMEM((_N * 4,), jnp.float32),
            pltpu.VMEM((_N * 4,), jnp.float32),
            pltpu.VMEM((64,), jnp.float32),
            pltpu.VMEM((_K,), jnp.int32),
            pltpu.VMEM((_K,), jnp.int32),
            pltpu.VMEM((_K, 16), jnp.float32),
            pltpu.VMEM((4 * _K,), jnp.float32),
            pltpu.VMEM_SHARED((_NP, 16), jnp.float32),
            pltpu.SemaphoreType.DMA,
        ],
    )
    return f(asrc, adst, ctbl, srcp, dstp)


def _make_sc1(p):
    def body(hcat, asrc, adst, ctbl, srcp, dstp, accm_out,
             asrc_v, adst_v, ctbl_v, src_b, src2_b, dst_b, rows_v, wtmp_v,
             accm_s, sem):
        c = lax.axis_index("c")
        s = lax.axis_index("s")
        cP = c * _NP
        hoff = (c * 2 + p) * _N
        myhdK = (c * 2 + p) * _K

        pltpu.sync_copy(asrc, asrc_v)
        pltpu.sync_copy(adst, adst_v)
        pltpu.sync_copy(ctbl, ctbl_v)
        _zero_acc(rows_v, None, accm_s, None, s, 4)

        tile_base = s * _ET1

        def _chunk(j, _):
            base = tile_base + j * _K
            pltpu.sync_copy(srcp.at[pl.ds(base, _K)], src_b)
            pltpu.sync_copy(dstp.at[pl.ds(base, _K)], dst_b)
            for g in range(8):
                src2_b[pl.ds(g * 16, 16)] = src_b[pl.ds(g * 16, 16)] + hoff
            pltpu.async_copy(hcat.at[src2_b], rows_v, sem).wait()
            _edge_w(asrc_v, adst_v, ctbl_v, src_b, dst_b, wtmp_v, base,
                    (p, p + 2), 4)

            def _edge(k, _):
                kv = jnp.full((16,), k, jnp.int32)
                sp = plsc.load_gather(wtmp_v, [kv + myhdK])
                for v in range(4):
                    rows_v[k, pl.ds(v * 16, 16)] = rows_v[k, pl.ds(v * 16, 16)] * sp
                return 0
            lax.fori_loop(0, _K, _edge, 0)

            pltpu.sync_copy(rows_v, accm_s.at[dst_b], add=True)
            return 0

        lax.fori_loop(0, _NC1, _chunk, 0)
        plsc.subcore_barrier()
        _write_acc(accm_s, rows_v, accm_out, s, cP)

    return pl.kernel(
        body, mesh=plsc.VectorSubcoreMesh(**_MESH),
        compiler_params=_SC_PARAMS,
        out_type=[jax.ShapeDtypeStruct((2 * _NP, 64), jnp.float32)],
        scratch_types=[
            pltpu.VMEM((_N * 4,), jnp.float32),
            pltpu.VMEM((_N * 4,), jnp.float32),
            pltpu.VMEM((64,), jnp.float32),
            pltpu.VMEM((_K,), jnp.int32),
            pltpu.VMEM((_K,), jnp.int32),
            pltpu.VMEM((_K,), jnp.int32),
            pltpu.VMEM((_K, 64), jnp.float32),
            pltpu.VMEM((4 * _K,), jnp.float32),
            pltpu.VMEM_SHARED((_NP, 64), jnp.float32),
            pltpu.SemaphoreType.DMA,
        ],
    )


def _sc2_body(h2, asrc2, adst2, ctbl2, srcp, dstp, accm_out, accd_out,
              asrc_v, adst_v, ctbl_v, src_b, src2_b, dst_b, rows_v, pad_v,
              wtmp_v, accm_s, accd_s, sem):
    c = lax.axis_index("c")
    s = lax.axis_index("s")
    cP = c * _NP
    cN = c * _N
    pltpu.sync_copy(asrc2, asrc_v)
    pltpu.sync_copy(adst2, adst_v)
    pltpu.sync_copy(ctbl2, ctbl_v)
    _zero_acc(rows_v, pad_v, accm_s, accd_s, s, 2)

    tile_base = s * _ET1
    lanes = _iota16()

    def _chunk(j, _):
        base = tile_base + j * _K
        pltpu.sync_copy(srcp.at[pl.ds(base, _K)], src_b)
        pltpu.sync_copy(dstp.at[pl.ds(base, _K)], dst_b)
        for g in range(8):
            src2_b[pl.ds(g * 16, 16)] = src_b[pl.ds(g * 16, 16)] + cN
        pltpu.async_copy(h2.at[src2_b], rows_v, sem).wait()

        for g in range(8):
            sv = src_b[pl.ds(g * 16, 16)]
            dv = dst_b[pl.ds(g * 16, 16)]
            eid = base + g * 16 + _iota16()
            e = plsc.load_gather(asrc_v, [sv]) + plsc.load_gather(adst_v, [dv])
            e = jnp.maximum(e, 0.2 * e) - ctbl_v[pl.ds(0, 16)]
            w = jnp.where(eid < _E, jnp.exp(e), 0.0)
            wtmp_v[pl.ds(g * 16, 16)] = w

        def _edge(k, _):
            kv = jnp.full((16,), k, jnp.int32)
            sp = plsc.load_gather(wtmp_v, [kv])
            for v in range(2):
                rows_v[k, pl.ds(v * 16, 16)] = rows_v[k, pl.ds(v * 16, 16)] * sp
            pad_v[k, :] = jnp.where(lanes < 1, sp, 0.0)
            return 0
        lax.fori_loop(0, _K, _edge, 0)

        pltpu.sync_copy(rows_v, accm_s.at[dst_b], add=True)
        pltpu.sync_copy(pad_v, accd_s.at[dst_b], add=True)
        return 0

    lax.fori_loop(0, _NC1, _chunk, 0)
    plsc.subcore_barrier()
    _write_acc(accm_s, rows_v, accm_out, s, cP)
    _write_acc(accd_s, pad_v, accd_out, s, cP)


def _sc2(h2, asrc2, adst2, ctbl2, srcp, dstp):
    f = pl.kernel(
        _sc2_body, mesh=plsc.VectorSubcoreMesh(**_MESH),
        compiler_params=_SC_PARAMS,
        out_type=[
            jax.ShapeDtypeStruct((2 * _NP, 32), jnp.float32),
            jax.ShapeDtypeStruct((2 * _NP, 16), jnp.float32),
        ],
        scratch_types=[
            pltpu.VMEM((_N,), jnp.float32),
            pltpu.VMEM((_N,), jnp.float32),
            pltpu.VMEM((16,), jnp.float32),
            pltpu.VMEM((_K,), jnp.int32),
            pltpu.VMEM((_K,), jnp.int32),
            pltpu.VMEM((_K,), jnp.int32),
            pltpu.VMEM((_K, 32), jnp.float32),
            pltpu.VMEM((_K, 16), jnp.float32),
            pltpu.VMEM((_K,), jnp.float32),
            pltpu.VMEM_SHARED((_NP, 32), jnp.float32),
            pltpu.VMEM_SHARED((_NP, 16), jnp.float32),
            pltpu.SemaphoreType.DMA,
        ],
    )
    return f(h2, asrc2, adst2, ctbl2, srcp, dstp)


def kernel(x, edge_index, W1, a_src1, a_dst1, b1, W2, a_src2, a_dst2, b2,
           W_fc, b_fc):
    a_s = jnp.zeros((256, _HEADS), jnp.float32)
    a_d = jnp.zeros((256, _HEADS), jnp.float32)
    for hd in range(_HEADS):
        a_s = a_s.at[hd * 64:(hd + 1) * 64, hd].set(a_src1[hd])
        a_d = a_d.at[hd * 64:(hd + 1) * 64, hd].set(a_dst1[hd])

    src = edge_index[0]
    dst = edge_index[1]
    npad = _EP - _E
    srcp = jnp.concatenate([src, jnp.zeros((npad,), jnp.int32)])
    dstp = jnp.concatenate([dst, (jnp.arange(npad, dtype=jnp.int32) * 97) % _N])

    hcat, asrc, adst, ctbl, root = _tc1(x, W1, a_s, a_d)
    asrc_f, adst_f, ctbl_f = asrc.reshape(-1), adst.reshape(-1), ctbl.reshape(-1)
    (accd1,) = _sc_den1(asrc_f, adst_f, ctbl_f, srcp, dstp)
    (accm_p0,) = _make_sc1(0)(hcat, asrc_f, adst_f, ctbl_f, srcp, dstp)
    (accm_p1,) = _make_sc1(1)(hcat, asrc_f, adst_f, ctbl_f, srcp, dstp)
    h2, asrc2, adst2 = _tc2(
        accm_p0[0:_N], accm_p1[0:_N], accm_p0[_NP:_NP + _N],
        accm_p1[_NP:_NP + _N], accd1[0:_N], accd1[_NP:_NP + _N],
        b1, W2, a_src2, a_dst2)
    ctbl2 = _tcc2(asrc2, adst2)
    h2cat = jnp.concatenate([h2[:, 0:32], h2[:, 32:64]], axis=0)
    accm2, accd2 = _sc2(h2cat, asrc2.reshape(-1), adst2.reshape(-1),
                        ctbl2.reshape(-1), srcp, dstp)
    out = _tc3(accm2[0:_N], accm2[_NP:_NP + _N], accd2[0:_N], b2, W_fc, b_fc)
    return out[root[0, 0]][None, :]


# speedup vs baseline: 21.3902x; 1.0251x over previous
import jax
import jax.numpy as jnp
from jax import lax
from jax.experimental import pallas as pl
from jax.experimental.pallas import tpu as pltpu
from jax.experimental.pallas import tpu_sc as plsc

_N = 10000
_E = 320000
_HEADS = 4

_K = 128
_NTILES = 16
_NCORES = 2

_EP = _NCORES * _NTILES * _K * ((_E + _NCORES * _NTILES * _K - 1) // (_NCORES * _NTILES * _K))
_ET1 = _EP // _NTILES
_NC1 = _ET1 // _K
_ET2 = _EP // (_NCORES * _NTILES)
_NC2 = _ET2 // _K

_NP = 10240
_ROWS_PER_TILE = _NP // _NTILES


def _iota16():
    return lax.iota(jnp.int32, 16)


def _tc1_body(x_ref, w1_ref, as_ref, ad_ref, hcat_ref, asrc_ref, adst_ref,
              ctbl_ref, root_ref):
    x = x_ref[...]
    h = jnp.dot(x, w1_ref[...], preferred_element_type=jnp.float32)
    for hd in range(4):
        hcat_ref[hd * _N:(hd + 1) * _N, :] = h[:, hd * 64:(hd + 1) * 64]
    asrc = jnp.dot(h, as_ref[...], preferred_element_type=jnp.float32)
    adst = jnp.dot(h, ad_ref[...], preferred_element_type=jnp.float32)
    asrc_ref[...] = asrc
    adst_ref[...] = adst
    c = jnp.maximum(jnp.max(asrc, axis=0) + jnp.max(adst, axis=0), 0.0)
    ctbl_ref[...] = jnp.broadcast_to(c[:, None], (4, 16))
    mask = x[:, 0:1] == 0.0
    ids = lax.broadcasted_iota(jnp.int32, (_N, 1), 0)
    rid = jnp.min(jnp.where(mask, ids, _N))
    root_ref[...] = jnp.broadcast_to(jnp.where(rid == _N, 0, rid), (1, 1))


def _tc1(x, w1, a_s, a_d):
    return pl.pallas_call(
        _tc1_body,
        out_shape=[
            jax.ShapeDtypeStruct((4 * _N, 64), jnp.float32),
            jax.ShapeDtypeStruct((_N, 4), jnp.float32),
            jax.ShapeDtypeStruct((_N, 4), jnp.float32),
            jax.ShapeDtypeStruct((4, 16), jnp.float32),
            jax.ShapeDtypeStruct((1, 1), jnp.int32),
        ],
    )(x, w1, a_s, a_d)


_B2 = 2000


def _tc2_body(a0l_ref, a1l_ref, a0h_ref, a1h_ref, adl_ref, adh_ref,
              b1_ref, w2_ref, as2_ref, ad2_ref,
              h2_ref, asrc2_ref, adst2_ref):
    den = adl_ref[:, 0:4] + adh_ref[:, 0:4] + 1e-16
    num = jnp.concatenate([a0l_ref[...], a1l_ref[...],
                           a0h_ref[...], a1h_ref[...]], axis=1)
    scale = jnp.concatenate(
        [jnp.broadcast_to(1.0 / den[:, hd:hd + 1], (_B2, 64)) for hd in range(4)],
        axis=1)
    h1 = jnp.maximum(num * scale + b1_ref[...][None, :], 0.0)
    h2 = jnp.dot(h1, w2_ref[...], preferred_element_type=jnp.float32)
    h2_ref[...] = h2
    asrc2_ref[...] = jnp.sum(h2 * as2_ref[...], axis=1, keepdims=True)
    adst2_ref[...] = jnp.sum(h2 * ad2_ref[...], axis=1, keepdims=True)


def _tc2(a0l, a1l, a0h, a1h, adl, adh, b1, w2, a_s2, a_d2):
    nblk = _N // _B2
    row = lambda i: (i, 0)
    full2 = lambda i: (0, 0)
    return pl.pallas_call(
        _tc2_body,
        grid=(nblk,),
        in_specs=[
            pl.BlockSpec((_B2, 64), row), pl.BlockSpec((_B2, 64), row),
            pl.BlockSpec((_B2, 64), row), pl.BlockSpec((_B2, 64), row),
            pl.BlockSpec((_B2, 16), row), pl.BlockSpec((_B2, 16), row),
            pl.BlockSpec((256,), lambda i: (0,)),
            pl.BlockSpec((256, 64), full2),
            pl.BlockSpec((1, 64), full2), pl.BlockSpec((1, 64), full2),
        ],
        out_specs=[
            pl.BlockSpec((_B2, 64), row),
            pl.BlockSpec((_B2, 1), row),
            pl.BlockSpec((_B2, 1), row),
        ],
        out_shape=[
            jax.ShapeDtypeStruct((_N, 64), jnp.float32),
            jax.ShapeDtypeStruct((_N, 1), jnp.float32),
            jax.ShapeDtypeStruct((_N, 1), jnp.float32),
        ],
    )(a0l, a1l, a0h, a1h, adl, adh, b1, w2, a_s2, a_d2)


def _tcc2_body(asrc2_ref, adst2_ref, ctbl2_ref):
    c2 = jnp.maximum(jnp.max(asrc2_ref[...]) + jnp.max(adst2_ref[...]), 0.0)
    ctbl2_ref[...] = jnp.full((1, 16), 1.0) * c2


def _tcc2(asrc2, adst2):
    return pl.pallas_call(
        _tcc2_body,
        out_shape=jax.ShapeDtypeStruct((1, 16), jnp.float32),
    )(asrc2, adst2)


def _tc3_body(m2l_ref, m2h_ref, d2l_ref, b2_ref, wfc_ref, bfc_ref, out_ref):
    num = jnp.concatenate([m2l_ref[...], m2h_ref[...]], axis=1)
    den = d2l_ref[:, 0:1] + 1e-16
    h2 = jnp.maximum(num / den + b2_ref[...][None, :], 0.0)
    out_ref[...] = (jnp.dot(h2, wfc_ref[...], preferred_element_type=jnp.float32)
                    + bfc_ref[...][None, :])


def _tc3(m2l, m2h, d2l, b2, wfc, bfc):
    return pl.pallas_call(
        _tc3_body,
        out_shape=jax.ShapeDtypeStruct((_N, 64), jnp.float32),
    )(m2l, m2h, d2l, b2, wfc, bfc)


_SC_PARAMS = pltpu.CompilerParams(needs_layout_passes=False,
                                  use_tc_tiling_on_sc=False)
_MESH = dict(core_axis_name="c", subcore_axis_name="s")


def _zero_acc(rows_v, pad_v, accm_s, accd_s, s, nvec):
    def _zrow(i, _):
        if rows_v is not None:
            for v in range(nvec):
                rows_v[i, pl.ds(v * 16, 16)] = jnp.zeros((16,), jnp.float32)
        if pad_v is not None:
            pad_v[i, :] = jnp.zeros((16,), jnp.float32)
        return 0
    lax.fori_loop(0, _K, _zrow, 0)
    for r in range(0, _ROWS_PER_TILE, _K):
        if accm_s is not None:
            pltpu.sync_copy(rows_v, accm_s.at[pl.ds(s * _ROWS_PER_TILE + r, _K), :])
        if accd_s is not None:
            pltpu.sync_copy(pad_v, accd_s.at[pl.ds(s * _ROWS_PER_TILE + r, _K), :])
    plsc.subcore_barrier()


def _write_acc(acc_s, buf_v, out_hbm, s, cP):
    for r in range(0, _ROWS_PER_TILE, _K):
        row0 = s * _ROWS_PER_TILE + r
        pltpu.sync_copy(acc_s.at[pl.ds(row0, _K), :], buf_v)
        pltpu.sync_copy(buf_v, out_hbm.at[pl.ds(cP + row0, _K), :])


def _edge_w(asrc_v, adst_v, ctbl_v, src_b, dst_b, wtmp_v, base, heads, nh):
    for g in range(8):
        sv = src_b[pl.ds(g * 16, 16)] * nh
        dv = dst_b[pl.ds(g * 16, 16)] * nh
        eid = base + g * 16 + _iota16()
        valid = eid < _E
        for hd in heads:
            e = (plsc.load_gather(asrc_v, [sv + hd])
                 + plsc.load_gather(adst_v, [dv + hd]))
            e = jnp.maximum(e, 0.2 * e) - ctbl_v[pl.ds(hd * 16, 16)]
            w = jnp.where(valid, jnp.exp(e), 0.0)
            wtmp_v[pl.ds(hd * _K + g * 16, 16)] = w


def _sc_den1_body(asrc, adst, ctbl, srcp, dstp, accd_out,
                  asrc_v, adst_v, ctbl_v, src_b, dst_b, pad_v, wtmp_v, accd_s,
                  sem):
    c = lax.axis_index("c")
    s = lax.axis_index("s")
    cP = c * _NP
    pltpu.sync_copy(asrc, asrc_v)
    pltpu.sync_copy(adst, adst_v)
    pltpu.sync_copy(ctbl, ctbl_v)
    _zero_acc(None, pad_v, None, accd_s, s, 0)

    tile_base = (c * _NTILES + s) * _ET2
    lanes = _iota16()

    def _chunk(j, _):
        base = tile_base + j * _K
        pltpu.sync_copy(srcp.at[pl.ds(base, _K)], src_b)
        pltpu.sync_copy(dstp.at[pl.ds(base, _K)], dst_b)
        _edge_w(asrc_v, adst_v, ctbl_v, src_b, dst_b, wtmp_v, base,
                (0, 1, 2, 3), 4)

        def _edge(k4, _):
            k = k4 * 4
            kv = (lanes & 3) * _K + jnp.full((16,), k, jnp.int32)
            for u in range(4):
                padv = plsc.load_gather(wtmp_v, [kv + u])
                pad_v[k + u, :] = jnp.where(lanes < 4, padv, 0.0)
            return 0
        lax.fori_loop(0, _K // 4, _edge, 0)

        pltpu.sync_copy(pad_v, accd_s.at[dst_b], add=True)
        return 0

    lax.fori_loop(0, _NC2, _chunk, 0)
    plsc.subcore_barrier()
    _write_acc(accd_s, pad_v, accd_out, s, cP)


def _sc_den1(asrc, adst, ctbl, srcp, dstp):
    f = pl.kernel(
        _sc_den1_body, mesh=plsc.VectorSubcoreMesh(**_MESH),
        compiler_params=_SC_PARAMS,
        out_type=[jax.ShapeDtypeStruct((2 * _NP, 16), jnp.float32)],
        scratch_types=[
            pltpu.VMEM((_N * 4,), jnp.float32),
            pltpu.VMEM((_N * 4,), jnp.float32),
            pltpu.VMEM((64,), jnp.float32),
            pltpu.VMEM((_K,), jnp.int32),
            pltpu.VMEM((_K,), jnp.int32),
            pltpu.VMEM((_K, 16), jnp.float32),
            pltpu.VMEM((4 * _K,), jnp.float32),
            pltpu.VMEM_SHARED((_NP, 16), jnp.float32),
            pltpu.SemaphoreType.DMA,
        ],
    )
    return f(asrc, adst, ctbl, srcp, dstp)


def _make_sc1(p):
    def body(hcat, asrc, adst, ctbl, srcp, dstp, accm_out,
             asrc_v, adst_v, ctbl_v, src_b, src2_b, dst_b, rows_v, wtmp_v,
             accm_s, sem):
        c = lax.axis_index("c")
        s = lax.axis_index("s")
        cP = c * _NP
        hoff = (c * 2 + p) * _N
        myhdK = (c * 2 + p) * _K

        pltpu.sync_copy(asrc, asrc_v)
        pltpu.sync_copy(adst, adst_v)
        pltpu.sync_copy(ctbl, ctbl_v)
        _zero_acc(rows_v, None, accm_s, None, s, 4)

        tile_base = s * _ET1

        def _chunk(j, _):
            base = tile_base + j * _K
            pltpu.sync_copy(srcp.at[pl.ds(base, _K)], src_b)
            pltpu.sync_copy(dstp.at[pl.ds(base, _K)], dst_b)
            for g in range(8):
                src2_b[pl.ds(g * 16, 16)] = src_b[pl.ds(g * 16, 16)] + hoff
            pltpu.async_copy(hcat.at[src2_b], rows_v, sem).wait()
            _edge_w(asrc_v, adst_v, ctbl_v, src_b, dst_b, wtmp_v, base,
                    (p, p + 2), 4)

            def _edge(k4, _):
                k = k4 * 4
                kv = jnp.full((16,), k, jnp.int32) + myhdK
                for u in range(4):
                    sp = plsc.load_gather(wtmp_v, [kv + u])
                    for v in range(4):
                        rows_v[k + u, pl.ds(v * 16, 16)] = (
                            rows_v[k + u, pl.ds(v * 16, 16)] * sp)
                return 0
            lax.fori_loop(0, _K // 4, _edge, 0)

            pltpu.sync_copy(rows_v, accm_s.at[dst_b], add=True)
            return 0

        lax.fori_loop(0, _NC1, _chunk, 0)
        plsc.subcore_barrier()
        _write_acc(accm_s, rows_v, accm_out, s, cP)

    return pl.kernel(
        body, mesh=plsc.VectorSubcoreMesh(**_MESH),
        compiler_params=_SC_PARAMS,
        out_type=[jax.ShapeDtypeStruct((2 * _NP, 64), jnp.float32)],
        scratch_types=[
            pltpu.VMEM((_N * 4,), jnp.float32),
            pltpu.VMEM((_N * 4,), jnp.float32),
            pltpu.VMEM((64,), jnp.float32),
            pltpu.VMEM((_K,), jnp.int32),
            pltpu.VMEM((_K,), jnp.int32),
            pltpu.VMEM((_K,), jnp.int32),
            pltpu.VMEM((_K, 64), jnp.float32),
            pltpu.VMEM((4 * _K,), jnp.float32),
            pltpu.VMEM_SHARED((_NP, 64), jnp.float32),
            pltpu.SemaphoreType.DMA,
        ],
    )


def _sc2_body(h2, asrc2, adst2, ctbl2, srcp, dstp, accm_out, accd_out,
              asrc_v, adst_v, ctbl_v, src_b, src2_b, dst_b, rows_v, pad_v,
              wtmp_v, accm_s, accd_s, sem):
    c = lax.axis_index("c")
    s = lax.axis_index("s")
    cP = c * _NP
    cN = c * _N
    pltpu.sync_copy(asrc2, asrc_v)
    pltpu.sync_copy(adst2, adst_v)
    pltpu.sync_copy(ctbl2, ctbl_v)
    _zero_acc(rows_v, pad_v, accm_s, accd_s, s, 2)

    tile_base = s * _ET1
    lanes = _iota16()

    def _chunk(j, _):
        base = tile_base + j * _K
        pltpu.sync_copy(srcp.at[pl.ds(base, _K)], src_b)
        pltpu.sync_copy(dstp.at[pl.ds(base, _K)], dst_b)
        for g in range(8):
            src2_b[pl.ds(g * 16, 16)] = src_b[pl.ds(g * 16, 16)] + cN
        pltpu.async_copy(h2.at[src2_b], rows_v, sem).wait()

        for g in range(8):
            sv = src_b[pl.ds(g * 16, 16)]
            dv = dst_b[pl.ds(g * 16, 16)]
            eid = base + g * 16 + _iota16()
            e = plsc.load_gather(asrc_v, [sv]) + plsc.load_gather(adst_v, [dv])
            e = jnp.maximum(e, 0.2 * e) - ctbl_v[pl.ds(0, 16)]
            w = jnp.where(eid < _E, jnp.exp(e), 0.0)
            wtmp_v[pl.ds(g * 16, 16)] = w

        def _edge(k4, _):
            k = k4 * 4
            kv = jnp.full((16,), k, jnp.int32)
            for u in range(4):
                sp = plsc.load_gather(wtmp_v, [kv + u])
                for v in range(2):
                    rows_v[k + u, pl.ds(v * 16, 16)] = (
                        rows_v[k + u, pl.ds(v * 16, 16)] * sp)
                pad_v[k + u, :] = jnp.where(lanes < 1, sp, 0.0)
            return 0
        lax.fori_loop(0, _K // 4, _edge, 0)

        pltpu.sync_copy(rows_v, accm_s.at[dst_b], add=True)
        pltpu.sync_copy(pad_v, accd_s.at[dst_b], add=True)
        return 0

    lax.fori_loop(0, _NC1, _chunk, 0)
    plsc.subcore_barrier()
    _write_acc(accm_s, rows_v, accm_out, s, cP)
    _write_acc(accd_s, pad_v, accd_out, s, cP)


def _sc2(h2, asrc2, adst2, ctbl2, srcp, dstp):
    f = pl.kernel(
        _sc2_body, mesh=plsc.VectorSubcoreMesh(**_MESH),
        compiler_params=_SC_PARAMS,
        out_type=[
            jax.ShapeDtypeStruct((2 * _NP, 32), jnp.float32),
            jax.ShapeDtypeStruct((2 * _NP, 16), jnp.float32),
        ],
        scratch_types=[
            pltpu.VMEM((_N,), jnp.float32),
            pltpu.VMEM((_N,), jnp.float32),
            pltpu.VMEM((16,), jnp.float32),
            pltpu.VMEM((_K,), jnp.int32),
            pltpu.VMEM((_K,), jnp.int32),
            pltpu.VMEM((_K,), jnp.int32),
            pltpu.VMEM((_K, 32), jnp.float32),
            pltpu.VMEM((_K, 16), jnp.float32),
            pltpu.VMEM((_K,), jnp.float32),
            pltpu.VMEM_SHARED((_NP, 32), jnp.float32),
            pltpu.VMEM_SHARED((_NP, 16), jnp.float32),
            pltpu.SemaphoreType.DMA,
        ],
    )
    return f(h2, asrc2, adst2, ctbl2, srcp, dstp)


def kernel(x, edge_index, W1, a_src1, a_dst1, b1, W2, a_src2, a_dst2, b2,
           W_fc, b_fc):
    a_s = jnp.zeros((256, _HEADS), jnp.float32)
    a_d = jnp.zeros((256, _HEADS), jnp.float32)
    for hd in range(_HEADS):
        a_s = a_s.at[hd * 64:(hd + 1) * 64, hd].set(a_src1[hd])
        a_d = a_d.at[hd * 64:(hd + 1) * 64, hd].set(a_dst1[hd])

    src = edge_index[0]
    dst = edge_index[1]
    npad = _EP - _E
    srcp = jnp.concatenate([src, jnp.zeros((npad,), jnp.int32)])
    dstp = jnp.concatenate([dst, (jnp.arange(npad, dtype=jnp.int32) * 97) % _N])

    hcat, asrc, adst, ctbl, root = _tc1(x, W1, a_s, a_d)
    asrc_f, adst_f, ctbl_f = asrc.reshape(-1), adst.reshape(-1), ctbl.reshape(-1)
    (accd1,) = _sc_den1(asrc_f, adst_f, ctbl_f, srcp, dstp)
    (accm_p0,) = _make_sc1(0)(hcat, asrc_f, adst_f, ctbl_f, srcp, dstp)
    (accm_p1,) = _make_sc1(1)(hcat, asrc_f, adst_f, ctbl_f, srcp, dstp)
    h2, asrc2, adst2 = _tc2(
        accm_p0[0:_N], accm_p1[0:_N], accm_p0[_NP:_NP + _N],
        accm_p1[_NP:_NP + _N], accd1[0:_N], accd1[_NP:_NP + _N],
        b1, W2, a_src2, a_dst2)
    ctbl2 = _tcc2(asrc2, adst2)
    h2cat = jnp.concatenate([h2[:, 0:32], h2[:, 32:64]], axis=0)
    accm2, accd2 = _sc2(h2cat, asrc2.reshape(-1), adst2.reshape(-1),
                        ctbl2.reshape(-1), srcp, dstp)
    out = _tc3(accm2[0:_N], accm2[_NP:_NP + _N], accd2[0:_N], b2, W_fc, b_fc)
    return out[root[0, 0]][None, :]


# speedup vs baseline: 24.2363x; 1.1331x over previous
import jax
import jax.numpy as jnp
from jax import lax
from jax.experimental import pallas as pl
from jax.experimental.pallas import tpu as pltpu
from jax.experimental.pallas import tpu_sc as plsc

_N = 10000
_E = 320000
_HEADS = 4

_K = 128
_NTILES = 16
_NCORES = 2

_EP = _NCORES * _NTILES * _K * ((_E + _NCORES * _NTILES * _K - 1) // (_NCORES * _NTILES * _K))
_ET1 = _EP // _NTILES
_NC1 = _ET1 // _K
_ET2 = _EP // (_NCORES * _NTILES)
_NC2 = _ET2 // _K

_NP = 10240
_ROWS_PER_TILE = _NP // _NTILES


def _iota16():
    return lax.iota(jnp.int32, 16)


def _tc1_body(x_ref, w1_ref, as_ref, ad_ref, hcat_ref, asrc_ref, adst_ref,
              ctbl_ref, root_ref):
    x = x_ref[...]
    h = jnp.dot(x, w1_ref[...], preferred_element_type=jnp.float32)
    for hd in range(4):
        hcat_ref[hd * _N:(hd + 1) * _N, :] = h[:, hd * 64:(hd + 1) * 64]
    asrc = jnp.dot(h, as_ref[...], preferred_element_type=jnp.float32)
    adst = jnp.dot(h, ad_ref[...], preferred_element_type=jnp.float32)
    asrc_ref[...] = asrc
    adst_ref[...] = adst
    c = jnp.maximum(jnp.max(asrc, axis=0) + jnp.max(adst, axis=0), 0.0)
    ctbl_ref[...] = jnp.broadcast_to(c[:, None], (4, 16))
    mask = x[:, 0:1] == 0.0
    ids = lax.broadcasted_iota(jnp.int32, (_N, 1), 0)
    rid = jnp.min(jnp.where(mask, ids, _N))
    root_ref[...] = jnp.broadcast_to(jnp.where(rid == _N, 0, rid), (1, 1))


def _tc1(x, w1, a_s, a_d):
    return pl.pallas_call(
        _tc1_body,
        out_shape=[
            jax.ShapeDtypeStruct((4 * _N, 64), jnp.float32),
            jax.ShapeDtypeStruct((_N, 4), jnp.float32),
            jax.ShapeDtypeStruct((_N, 4), jnp.float32),
            jax.ShapeDtypeStruct((4, 16), jnp.float32),
            jax.ShapeDtypeStruct((1, 1), jnp.int32),
        ],
    )(x, w1, a_s, a_d)


_B2 = 2000


def _tc2_body(a0l_ref, a1l_ref, a0h_ref, a1h_ref, adl_ref, adh_ref,
              b1_ref, w2_ref, as2_ref, ad2_ref,
              h2_ref, asrc2_ref, adst2_ref):
    den = adl_ref[:, 0:4] + adh_ref[:, 0:4] + 1e-16
    num = jnp.concatenate([a0l_ref[...], a1l_ref[...],
                           a0h_ref[...], a1h_ref[...]], axis=1)
    scale = jnp.concatenate(
        [jnp.broadcast_to(1.0 / den[:, hd:hd + 1], (_B2, 64)) for hd in range(4)],
        axis=1)
    h1 = jnp.maximum(num * scale + b1_ref[...][None, :], 0.0)
    h2 = jnp.dot(h1, w2_ref[...], preferred_element_type=jnp.float32)
    h2_ref[...] = h2
    asrc2_ref[...] = jnp.sum(h2 * as2_ref[...], axis=1, keepdims=True)
    adst2_ref[...] = jnp.sum(h2 * ad2_ref[...], axis=1, keepdims=True)


def _tc2(a0l, a1l, a0h, a1h, adl, adh, b1, w2, a_s2, a_d2):
    nblk = _N // _B2
    row = lambda i: (i, 0)
    full2 = lambda i: (0, 0)
    return pl.pallas_call(
        _tc2_body,
        grid=(nblk,),
        in_specs=[
            pl.BlockSpec((_B2, 64), row), pl.BlockSpec((_B2, 64), row),
            pl.BlockSpec((_B2, 64), row), pl.BlockSpec((_B2, 64), row),
            pl.BlockSpec((_B2, 16), row), pl.BlockSpec((_B2, 16), row),
            pl.BlockSpec((256,), lambda i: (0,)),
            pl.BlockSpec((256, 64), full2),
            pl.BlockSpec((1, 64), full2), pl.BlockSpec((1, 64), full2),
        ],
        out_specs=[
            pl.BlockSpec((_B2, 64), row),
            pl.BlockSpec((_B2, 1), row),
            pl.BlockSpec((_B2, 1), row),
        ],
        out_shape=[
            jax.ShapeDtypeStruct((_N, 64), jnp.float32),
            jax.ShapeDtypeStruct((_N, 1), jnp.float32),
            jax.ShapeDtypeStruct((_N, 1), jnp.float32),
        ],
    )(a0l, a1l, a0h, a1h, adl, adh, b1, w2, a_s2, a_d2)


def _tcc2_body(asrc2_ref, adst2_ref, ctbl2_ref):
    c2 = jnp.maximum(jnp.max(asrc2_ref[...]) + jnp.max(adst2_ref[...]), 0.0)
    ctbl2_ref[...] = jnp.full((1, 16), 1.0) * c2


def _tcc2(asrc2, adst2):
    return pl.pallas_call(
        _tcc2_body,
        out_shape=jax.ShapeDtypeStruct((1, 16), jnp.float32),
    )(asrc2, adst2)


def _tc3_body(m2l_ref, m2h_ref, d2l_ref, b2_ref, wfc_ref, bfc_ref, out_ref):
    num = jnp.concatenate([m2l_ref[...], m2h_ref[...]], axis=1)
    den = d2l_ref[:, 0:1] + 1e-16
    h2 = jnp.maximum(num / den + b2_ref[...][None, :], 0.0)
    out_ref[...] = (jnp.dot(h2, wfc_ref[...], preferred_element_type=jnp.float32)
                    + bfc_ref[...][None, :])


def _tc3(m2l, m2h, d2l, b2, wfc, bfc):
    return pl.pallas_call(
        _tc3_body,
        out_shape=jax.ShapeDtypeStruct((_N, 64), jnp.float32),
    )(m2l, m2h, d2l, b2, wfc, bfc)


_SC_PARAMS = pltpu.CompilerParams(needs_layout_passes=False,
                                  use_tc_tiling_on_sc=False)
_MESH = dict(core_axis_name="c", subcore_axis_name="s")


def _zero_acc(rows_v, pad_v, accm_s, accd_s, s, nvec):
    def _zrow(i, _):
        if rows_v is not None:
            for v in range(nvec):
                rows_v[i, pl.ds(v * 16, 16)] = jnp.zeros((16,), jnp.float32)
        if pad_v is not None:
            pad_v[i, :] = jnp.zeros((16,), jnp.float32)
        return 0
    lax.fori_loop(0, _K, _zrow, 0)
    for r in range(0, _ROWS_PER_TILE, _K):
        if accm_s is not None:
            pltpu.sync_copy(rows_v, accm_s.at[pl.ds(s * _ROWS_PER_TILE + r, _K), :])
        if accd_s is not None:
            pltpu.sync_copy(pad_v, accd_s.at[pl.ds(s * _ROWS_PER_TILE + r, _K), :])
    plsc.subcore_barrier()


def _write_acc(acc_s, buf_v, out_hbm, s, cP):
    for r in range(0, _ROWS_PER_TILE, _K):
        row0 = s * _ROWS_PER_TILE + r
        pltpu.sync_copy(acc_s.at[pl.ds(row0, _K), :], buf_v)
        pltpu.sync_copy(buf_v, out_hbm.at[pl.ds(cP + row0, _K), :])


def _edge_w(asrc_v, adst_v, ctbl_v, src_b, dst_b, wtmp_v, base, heads, nh):
    for g in range(8):
        sv = src_b[pl.ds(g * 16, 16)] * nh
        dv = dst_b[pl.ds(g * 16, 16)] * nh
        eid = base + g * 16 + _iota16()
        valid = eid < _E
        for hd in heads:
            e = (plsc.load_gather(asrc_v, [sv + hd])
                 + plsc.load_gather(adst_v, [dv + hd]))
            e = jnp.maximum(e, 0.2 * e) - ctbl_v[pl.ds(hd * 16, 16)]
            w = jnp.where(valid, jnp.exp(e), 0.0)
            wtmp_v[pl.ds(hd * _K + g * 16, 16)] = w


def _edge_w2(asrc_v, adst_v, ctbl_v, src_b, dst_b, wtmp_v, base):
    for g in range(8):
        sv = src_b[pl.ds(g * 16, 16)]
        dv = dst_b[pl.ds(g * 16, 16)]
        eid = base + g * 16 + _iota16()
        valid = eid < _E
        for slot in range(2):
            e = (plsc.load_gather(asrc_v, [sv + slot * _N])
                 + plsc.load_gather(adst_v, [dv + slot * _N]))
            e = jnp.maximum(e, 0.2 * e) - ctbl_v[pl.ds(slot * 16, 16)]
            w = jnp.where(valid, jnp.exp(e), 0.0)
            wtmp_v[pl.ds(slot * _K + g * 16, 16)] = w


def _sc_den1_body(asrc, adst, ctbl, srcp, dstp, accd_out,
                  asrc_v, adst_v, ctbl_v, src_b, dst_b, pad_v, wtmp_v, accd_s,
                  sem):
    c = lax.axis_index("c")
    s = lax.axis_index("s")
    cP = c * _NP
    pltpu.sync_copy(asrc, asrc_v)
    pltpu.sync_copy(adst, adst_v)
    pltpu.sync_copy(ctbl, ctbl_v)
    _zero_acc(None, pad_v, None, accd_s, s, 0)

    tile_base = (c * _NTILES + s) * _ET2
    lanes = _iota16()

    tile_row = (c * _NTILES + s) * _NC2

    def _chunk(j, _):
        base = tile_base + j * _K
        pltpu.sync_copy(srcp.at[tile_row + j], src_b)
        pltpu.sync_copy(dstp.at[tile_row + j], dst_b)
        _edge_w(asrc_v, adst_v, ctbl_v, src_b, dst_b, wtmp_v, base,
                (0, 1, 2, 3), 4)

        def _edge(k4, _):
            k = k4 * 4
            kv = (lanes & 3) * _K + jnp.full((16,), k, jnp.int32)
            for u in range(4):
                padv = plsc.load_gather(wtmp_v, [kv + u])
                pad_v[k + u, :] = jnp.where(lanes < 4, padv, 0.0)
            return 0
        lax.fori_loop(0, _K // 4, _edge, 0)

        pltpu.sync_copy(pad_v, accd_s.at[dst_b], add=True)
        return 0

    lax.fori_loop(0, _NC2, _chunk, 0)
    plsc.subcore_barrier()
    _write_acc(accd_s, pad_v, accd_out, s, cP)


def _sc_den1(asrc, adst, ctbl, srcp, dstp):
    f = pl.kernel(
        _sc_den1_body, mesh=plsc.VectorSubcoreMesh(**_MESH),
        compiler_params=_SC_PARAMS,
        out_type=[jax.ShapeDtypeStruct((2 * _NP, 16), jnp.float32)],
        scratch_types=[
            pltpu.VMEM((_N * 4,), jnp.float32),
            pltpu.VMEM((_N * 4,), jnp.float32),
            pltpu.VMEM((64,), jnp.float32),
            pltpu.VMEM((_K,), jnp.int32),
            pltpu.VMEM((_K,), jnp.int32),
            pltpu.VMEM((_K, 16), jnp.float32),
            pltpu.VMEM((4 * _K,), jnp.float32),
            pltpu.VMEM_SHARED((_NP, 16), jnp.float32),
            pltpu.SemaphoreType.DMA,
        ],
    )
    return f(asrc, adst, ctbl, srcp, dstp)


def _make_sc1(p):
    def body(hcat, asrc, adst, ctbl, srcp, dstp, accm_out,
             asrc_v, adst_v, ctbl_v, src_b, dst_b, src2_b, dsc_b,
             rows0, rows1, wtmp_v, accm_s, gsem0, gsem1):
        c = lax.axis_index("c")
        s = lax.axis_index("s")
        cP = c * _NP
        hoff = (c * 2 + p) * _N
        myhdK = c * _K

        pltpu.sync_copy(asrc, asrc_v)
        pltpu.sync_copy(adst, adst_v)
        pltpu.sync_copy(ctbl, ctbl_v)
        _zero_acc(rows0, None, accm_s, None, s, 4)

        tile_base = s * _ET1

        def _mul_rows(rows_v):
            def _edge(k4, _):
                k = k4 * 4
                kv = jnp.full((16,), k, jnp.int32) + myhdK
                for u in range(4):
                    sp = plsc.load_gather(wtmp_v, [kv + u])
                    for v in range(4):
                        rows_v[k + u, pl.ds(v * 16, 16)] = (
                            rows_v[k + u, pl.ds(v * 16, 16)] * sp)
                return 0
            lax.fori_loop(0, _K // 4, _edge, 0)

        def _drain_rows(rows_v, sem):
            pltpu.make_async_copy(accm_out.at[pl.ds(0, _K), :], rows_v, sem).wait()

        def _pipe(j, rowsA, gsemA, rowsB, gsemB):
            base = tile_base + j * _K
            pltpu.sync_copy(srcp.at[tile_row + j + 1], src_b)
            pltpu.sync_copy(dstp.at[tile_row + j + 1], dst_b)
            _drain_rows(rowsA, gsemA)
            _mul_rows(rowsA)
            pltpu.sync_copy(rowsA, accm_s.at[dsc_b], add=True)
            for g in range(8):
                src2_b[pl.ds(g * 16, 16)] = src_b[pl.ds(g * 16, 16)] + hoff
            pltpu.async_copy(hcat.at[src2_b], rowsB, gsemB)
            _edge_w2(asrc_v, adst_v, ctbl_v, src_b, dst_b, wtmp_v, base + _K)
            for g in range(8):
                dsc_b[pl.ds(g * 16, 16)] = dst_b[pl.ds(g * 16, 16)]

        tile_row = s * _NC1
        pltpu.sync_copy(srcp.at[tile_row], src_b)
        pltpu.sync_copy(dstp.at[tile_row], dst_b)
        for g in range(8):
            src2_b[pl.ds(g * 16, 16)] = src_b[pl.ds(g * 16, 16)] + hoff
        pltpu.async_copy(hcat.at[src2_b], rows0, gsem0)
        _edge_w2(asrc_v, adst_v, ctbl_v, src_b, dst_b, wtmp_v, tile_base)
        for g in range(8):
            dsc_b[pl.ds(g * 16, 16)] = dst_b[pl.ds(g * 16, 16)]

        def _pair(g2, _):
            j0 = g2 * 2
            _pipe(j0, rows0, gsem0, rows1, gsem1)
            _pipe(j0 + 1, rows1, gsem1, rows0, gsem0)
            return 0
        lax.fori_loop(0, _NC1 // 2, _pair, 0)
        _drain_rows(rows0, gsem0)

        plsc.subcore_barrier()
        _write_acc(accm_s, rows0, accm_out, s, cP)

    return pl.kernel(
        body, mesh=plsc.VectorSubcoreMesh(**_MESH),
        compiler_params=_SC_PARAMS,
        out_type=[jax.ShapeDtypeStruct((2 * _NP, 64), jnp.float32)],
        scratch_types=[
            pltpu.VMEM((_N * 2,), jnp.float32),
            pltpu.VMEM((_N * 2,), jnp.float32),
            pltpu.VMEM((32,), jnp.float32),
            pltpu.VMEM((_K,), jnp.int32),
            pltpu.VMEM((_K,), jnp.int32),
            pltpu.VMEM((_K,), jnp.int32),
            pltpu.VMEM((_K,), jnp.int32),
            pltpu.VMEM((_K, 64), jnp.float32),
            pltpu.VMEM((_K, 64), jnp.float32),
            pltpu.VMEM((2 * _K,), jnp.float32),
            pltpu.VMEM_SHARED((_NP, 64), jnp.float32),
            pltpu.SemaphoreType.DMA,
            pltpu.SemaphoreType.DMA,
        ],
    )


def _sc2_body(h2, asrc2, adst2, ctbl2, srcp, dstp, accm_out, accd_out,
              asrc_v, adst_v, ctbl_v, src_b, src2_b, dst_b, rows_v, pad_v,
              wtmp_v, accm_s, accd_s, sem):
    c = lax.axis_index("c")
    s = lax.axis_index("s")
    cP = c * _NP
    cN = c * _N
    pltpu.sync_copy(asrc2, asrc_v)
    pltpu.sync_copy(adst2, adst_v)
    pltpu.sync_copy(ctbl2, ctbl_v)
    _zero_acc(rows_v, pad_v, accm_s, accd_s, s, 2)

    tile_base = s * _ET1
    lanes = _iota16()

    tile_row = s * _NC1

    def _chunk(j, _):
        base = tile_base + j * _K
        pltpu.sync_copy(srcp.at[tile_row + j], src_b)
        pltpu.sync_copy(dstp.at[tile_row + j], dst_b)
        for g in range(8):
            src2_b[pl.ds(g * 16, 16)] = src_b[pl.ds(g * 16, 16)] + cN
        pltpu.async_copy(h2.at[src2_b], rows_v, sem).wait()

        for g in range(8):
            sv = src_b[pl.ds(g * 16, 16)]
            dv = dst_b[pl.ds(g * 16, 16)]
            eid = base + g * 16 + _iota16()
            e = plsc.load_gather(asrc_v, [sv]) + plsc.load_gather(adst_v, [dv])
            e = jnp.maximum(e, 0.2 * e) - ctbl_v[pl.ds(0, 16)]
            w = jnp.where(eid < _E, jnp.exp(e), 0.0)
            wtmp_v[pl.ds(g * 16, 16)] = w

        def _edge(k4, _):
            k = k4 * 4
            kv = jnp.full((16,), k, jnp.int32)
            for u in range(4):
                sp = plsc.load_gather(wtmp_v, [kv + u])
                for v in range(2):
                    rows_v[k + u, pl.ds(v * 16, 16)] = (
                        rows_v[k + u, pl.ds(v * 16, 16)] * sp)
                pad_v[k + u, :] = jnp.where(lanes < 1, sp, 0.0)
            return 0
        lax.fori_loop(0, _K // 4, _edge, 0)

        pltpu.sync_copy(rows_v, accm_s.at[dst_b], add=True)
        pltpu.sync_copy(pad_v, accd_s.at[dst_b], add=True)
        return 0

    lax.fori_loop(0, _NC1, _chunk, 0)
    plsc.subcore_barrier()
    _write_acc(accm_s, rows_v, accm_out, s, cP)
    _write_acc(accd_s, pad_v, accd_out, s, cP)


def _sc2(h2, asrc2, adst2, ctbl2, srcp, dstp):
    f = pl.kernel(
        _sc2_body, mesh=plsc.VectorSubcoreMesh(**_MESH),
        compiler_params=_SC_PARAMS,
        out_type=[
            jax.ShapeDtypeStruct((2 * _NP, 32), jnp.float32),
            jax.ShapeDtypeStruct((2 * _NP, 16), jnp.float32),
        ],
        scratch_types=[
            pltpu.VMEM((_N,), jnp.float32),
            pltpu.VMEM((_N,), jnp.float32),
            pltpu.VMEM((16,), jnp.float32),
            pltpu.VMEM((_K,), jnp.int32),
            pltpu.VMEM((_K,), jnp.int32),
            pltpu.VMEM((_K,), jnp.int32),
            pltpu.VMEM((_K, 32), jnp.float32),
            pltpu.VMEM((_K, 16), jnp.float32),
            pltpu.VMEM((_K,), jnp.float32),
            pltpu.VMEM_SHARED((_NP, 32), jnp.float32),
            pltpu.VMEM_SHARED((_NP, 16), jnp.float32),
            pltpu.SemaphoreType.DMA,
        ],
    )
    return f(h2, asrc2, adst2, ctbl2, srcp, dstp)


def kernel(x, edge_index, W1, a_src1, a_dst1, b1, W2, a_src2, a_dst2, b2,
           W_fc, b_fc):
    a_s = jnp.zeros((256, _HEADS), jnp.float32)
    a_d = jnp.zeros((256, _HEADS), jnp.float32)
    for hd in range(_HEADS):
        a_s = a_s.at[hd * 64:(hd + 1) * 64, hd].set(a_src1[hd])
        a_d = a_d.at[hd * 64:(hd + 1) * 64, hd].set(a_dst1[hd])

    src = edge_index[0]
    dst = edge_index[1]
    npad = _EP + 2 * _K - _E
    srcp = jnp.concatenate([src, jnp.zeros((npad,), jnp.int32)]).reshape(-1, _K)
    dstp = jnp.concatenate(
        [dst, (jnp.arange(npad, dtype=jnp.int32) * 97) % _N]).reshape(-1, _K)

    hcat, asrc, adst, ctbl, root = _tc1(x, W1, a_s, a_d)
    asrc_f, adst_f, ctbl_f = asrc.reshape(-1), adst.reshape(-1), ctbl.reshape(-1)
    (accd1,) = _sc_den1(asrc_f, adst_f, ctbl_f, srcp, dstp)
    asrc_t, adst_t = asrc.T, adst.T
    accm_p = []
    for pp in range(2):
        a_p = jnp.concatenate([asrc_t[pp], asrc_t[pp + 2]])
        d_p = jnp.concatenate([adst_t[pp], adst_t[pp + 2]])
        c_p = jnp.concatenate([ctbl[pp], ctbl[pp + 2]])
        accm_p.append(_make_sc1(pp)(hcat, a_p, d_p, c_p, srcp, dstp)[0])
    accm_p0, accm_p1 = accm_p
    h2, asrc2, adst2 = _tc2(
        accm_p0[0:_N], accm_p1[0:_N], accm_p0[_NP:_NP + _N],
        accm_p1[_NP:_NP + _N], accd1[0:_N], accd1[_NP:_NP + _N],
        b1, W2, a_src2, a_dst2)
    ctbl2 = _tcc2(asrc2, adst2)
    h2cat = jnp.concatenate([h2[:, 0:32], h2[:, 32:64]], axis=0)
    accm2, accd2 = _sc2(h2cat, asrc2.reshape(-1), adst2.reshape(-1),
                        ctbl2.reshape(-1), srcp, dstp)
    out = _tc3(accm2[0:_N], accm2[_NP:_NP + _N], accd2[0:_N], b2, W_fc, b_fc)
    return out[root[0, 0]][None, :]


# speedup vs baseline: 24.9336x; 1.0288x over previous
import jax
import jax.numpy as jnp
from jax import lax
from jax.experimental import pallas as pl
from jax.experimental.pallas import tpu as pltpu
from jax.experimental.pallas import tpu_sc as plsc

_N = 10000
_E = 320000
_HEADS = 4

_K = 128
_NTILES = 16
_NCORES = 2

_EP = _NCORES * _NTILES * _K * ((_E + _NCORES * _NTILES * _K - 1) // (_NCORES * _NTILES * _K))
_ET1 = _EP // _NTILES
_NC1 = _ET1 // _K
_ET2 = _EP // (_NCORES * _NTILES)
_NC2 = _ET2 // _K

_NP = 10240
_ROWS_PER_TILE = _NP // _NTILES


def _iota16():
    return lax.iota(jnp.int32, 16)


def _tc1_body(x_ref, w1_ref, as_ref, ad_ref, hcat_ref, asrc_ref, adst_ref,
              ctbl_ref, root_ref):
    x = x_ref[...]
    h = jnp.dot(x, w1_ref[...], preferred_element_type=jnp.float32)
    for hd in range(4):
        hcat_ref[hd * _N:(hd + 1) * _N, :] = h[:, hd * 64:(hd + 1) * 64]
    asrc = jnp.dot(h, as_ref[...], preferred_element_type=jnp.float32)
    adst = jnp.dot(h, ad_ref[...], preferred_element_type=jnp.float32)
    asrc_ref[...] = asrc
    adst_ref[...] = adst
    c = jnp.maximum(jnp.max(asrc, axis=0) + jnp.max(adst, axis=0), 0.0)
    ctbl_ref[...] = jnp.broadcast_to(c[:, None], (4, 16))
    mask = x[:, 0:1] == 0.0
    ids = lax.broadcasted_iota(jnp.int32, (_N, 1), 0)
    rid = jnp.min(jnp.where(mask, ids, _N))
    root_ref[...] = jnp.broadcast_to(jnp.where(rid == _N, 0, rid), (1, 1))


def _tc1(x, w1, a_s, a_d):
    return pl.pallas_call(
        _tc1_body,
        out_shape=[
            jax.ShapeDtypeStruct((4 * _N, 64), jnp.float32),
            jax.ShapeDtypeStruct((_N, 4), jnp.float32),
            jax.ShapeDtypeStruct((_N, 4), jnp.float32),
            jax.ShapeDtypeStruct((4, 16), jnp.float32),
            jax.ShapeDtypeStruct((1, 1), jnp.int32),
        ],
    )(x, w1, a_s, a_d)


_B2 = 2000


def _tc2_body(a0l_ref, a1l_ref, a0h_ref, a1h_ref, adl_ref, adh_ref,
              b1_ref, w2_ref, as2_ref, ad2_ref,
              h2_ref, asrc2_ref, adst2_ref):
    den = adl_ref[:, 0:4] + adh_ref[:, 0:4] + 1e-16
    num = jnp.concatenate([a0l_ref[...], a1l_ref[...],
                           a0h_ref[...], a1h_ref[...]], axis=1)
    scale = jnp.concatenate(
        [jnp.broadcast_to(1.0 / den[:, hd:hd + 1], (_B2, 64)) for hd in range(4)],
        axis=1)
    h1 = jnp.maximum(num * scale + b1_ref[...][None, :], 0.0)
    h2 = jnp.dot(h1, w2_ref[...], preferred_element_type=jnp.float32)
    h2_ref[...] = h2
    asrc2_ref[...] = jnp.sum(h2 * as2_ref[...], axis=1, keepdims=True)
    adst2_ref[...] = jnp.sum(h2 * ad2_ref[...], axis=1, keepdims=True)


def _tc2(a0l, a1l, a0h, a1h, adl, adh, b1, w2, a_s2, a_d2):
    nblk = _N // _B2
    row = lambda i: (i, 0)
    full2 = lambda i: (0, 0)
    return pl.pallas_call(
        _tc2_body,
        grid=(nblk,),
        in_specs=[
            pl.BlockSpec((_B2, 64), row), pl.BlockSpec((_B2, 64), row),
            pl.BlockSpec((_B2, 64), row), pl.BlockSpec((_B2, 64), row),
            pl.BlockSpec((_B2, 16), row), pl.BlockSpec((_B2, 16), row),
            pl.BlockSpec((256,), lambda i: (0,)),
            pl.BlockSpec((256, 64), full2),
            pl.BlockSpec((1, 64), full2), pl.BlockSpec((1, 64), full2),
        ],
        out_specs=[
            pl.BlockSpec((_B2, 64), row),
            pl.BlockSpec((_B2, 1), row),
            pl.BlockSpec((_B2, 1), row),
        ],
        out_shape=[
            jax.ShapeDtypeStruct((_N, 64), jnp.float32),
            jax.ShapeDtypeStruct((_N, 1), jnp.float32),
            jax.ShapeDtypeStruct((_N, 1), jnp.float32),
        ],
    )(a0l, a1l, a0h, a1h, adl, adh, b1, w2, a_s2, a_d2)


def _tcc2_body(asrc2_ref, adst2_ref, ctbl2_ref):
    c2 = jnp.maximum(jnp.max(asrc2_ref[...]) + jnp.max(adst2_ref[...]), 0.0)
    ctbl2_ref[...] = jnp.full((1, 16), 1.0) * c2


def _tcc2(asrc2, adst2):
    return pl.pallas_call(
        _tcc2_body,
        out_shape=jax.ShapeDtypeStruct((1, 16), jnp.float32),
    )(asrc2, adst2)


def _tc3_body(m2l_ref, m2h_ref, d2l_ref, b2_ref, wfc_ref, bfc_ref, out_ref):
    num = jnp.concatenate([m2l_ref[...], m2h_ref[...]], axis=1)
    den = d2l_ref[:, 0:1] + 1e-16
    h2 = jnp.maximum(num / den + b2_ref[...][None, :], 0.0)
    out_ref[...] = (jnp.dot(h2, wfc_ref[...], preferred_element_type=jnp.float32)
                    + bfc_ref[...][None, :])


def _tc3(m2l, m2h, d2l, b2, wfc, bfc):
    return pl.pallas_call(
        _tc3_body,
        out_shape=jax.ShapeDtypeStruct((_N, 64), jnp.float32),
    )(m2l, m2h, d2l, b2, wfc, bfc)


_SC_PARAMS = pltpu.CompilerParams(needs_layout_passes=False,
                                  use_tc_tiling_on_sc=False)
_MESH = dict(core_axis_name="c", subcore_axis_name="s")


def _zero_acc(rows_v, pad_v, accm_s, accd_s, s, nvec):
    def _zrow(i, _):
        if rows_v is not None:
            for v in range(nvec):
                rows_v[i, pl.ds(v * 16, 16)] = jnp.zeros((16,), jnp.float32)
        if pad_v is not None:
            pad_v[i, :] = jnp.zeros((16,), jnp.float32)
        return 0
    lax.fori_loop(0, _K, _zrow, 0)
    for r in range(0, _ROWS_PER_TILE, _K):
        if accm_s is not None:
            pltpu.sync_copy(rows_v, accm_s.at[pl.ds(s * _ROWS_PER_TILE + r, _K), :])
        if accd_s is not None:
            pltpu.sync_copy(pad_v, accd_s.at[pl.ds(s * _ROWS_PER_TILE + r, _K), :])
    plsc.subcore_barrier()


def _write_acc(acc_s, buf_v, out_hbm, s, cP):
    for r in range(0, _ROWS_PER_TILE, _K):
        row0 = s * _ROWS_PER_TILE + r
        pltpu.sync_copy(acc_s.at[pl.ds(row0, _K), :], buf_v)
        pltpu.sync_copy(buf_v, out_hbm.at[pl.ds(cP + row0, _K), :])


def _edge_w(asrc_v, adst_v, ctbl_v, src_b, dst_b, wtmp_v, base, heads, nh):
    for g in range(8):
        sv = src_b[pl.ds(g * 16, 16)] * nh
        dv = dst_b[pl.ds(g * 16, 16)] * nh
        eid = base + g * 16 + _iota16()
        valid = eid < _E
        for hd in heads:
            e = (plsc.load_gather(asrc_v, [sv + hd])
                 + plsc.load_gather(adst_v, [dv + hd]))
            e = jnp.maximum(e, 0.2 * e) - ctbl_v[pl.ds(hd * 16, 16)]
            w = jnp.where(valid, jnp.exp(e), 0.0)
            wtmp_v[pl.ds(hd * _K + g * 16, 16)] = w


def _edge_w2(asrc_v, adst_v, ctbl_v, src_b, dst_b, wtmp_v, base):
    for g in range(8):
        sv = src_b[pl.ds(g * 16, 16)]
        dv = dst_b[pl.ds(g * 16, 16)]
        eid = base + g * 16 + _iota16()
        valid = eid < _E
        for slot in range(2):
            e = (plsc.load_gather(asrc_v, [sv + slot * _N])
                 + plsc.load_gather(adst_v, [dv + slot * _N]))
            e = jnp.maximum(e, 0.2 * e) - ctbl_v[pl.ds(slot * 16, 16)]
            w = jnp.where(valid, jnp.exp(e), 0.0)
            wtmp_v[pl.ds(slot * _K + g * 16, 16)] = w


def _sc_den1_body(asrc, adst, ctbl, srcp, dstp, accd_out,
                  asrc_v, adst_v, ctbl_v, src_b, dst_b, pad_v, wtmp_v, accd_s,
                  sem):
    c = lax.axis_index("c")
    s = lax.axis_index("s")
    cP = c * _NP
    pltpu.sync_copy(asrc, asrc_v)
    pltpu.sync_copy(adst, adst_v)
    pltpu.sync_copy(ctbl, ctbl_v)
    _zero_acc(None, pad_v, None, accd_s, s, 0)

    tile_base = (c * _NTILES + s) * _ET2
    lanes = _iota16()

    tile_row = (c * _NTILES + s) * _NC2

    def _chunk(j, _):
        base = tile_base + j * _K
        pltpu.sync_copy(srcp.at[tile_row + j], src_b)
        pltpu.sync_copy(dstp.at[tile_row + j], dst_b)
        _edge_w(asrc_v, adst_v, ctbl_v, src_b, dst_b, wtmp_v, base,
                (0, 1, 2, 3), 4)

        def _edge(k4, _):
            k = k4 * 4
            kv = (lanes & 3) * _K + jnp.full((16,), k, jnp.int32)
            for u in range(4):
                padv = plsc.load_gather(wtmp_v, [kv + u])
                pad_v[k + u, :] = jnp.where(lanes < 4, padv, 0.0)
            return 0
        lax.fori_loop(0, _K // 4, _edge, 0)

        pltpu.sync_copy(pad_v, accd_s.at[dst_b], add=True)
        return 0

    lax.fori_loop(0, _NC2, _chunk, 0)
    plsc.subcore_barrier()
    _write_acc(accd_s, pad_v, accd_out, s, cP)


def _sc_den1(asrc, adst, ctbl, srcp, dstp):
    f = pl.kernel(
        _sc_den1_body, mesh=plsc.VectorSubcoreMesh(**_MESH),
        compiler_params=_SC_PARAMS,
        out_type=[jax.ShapeDtypeStruct((2 * _NP, 16), jnp.float32)],
        scratch_types=[
            pltpu.VMEM((_N * 4,), jnp.float32),
            pltpu.VMEM((_N * 4,), jnp.float32),
            pltpu.VMEM((64,), jnp.float32),
            pltpu.VMEM((_K,), jnp.int32),
            pltpu.VMEM((_K,), jnp.int32),
            pltpu.VMEM((_K, 16), jnp.float32),
            pltpu.VMEM((4 * _K,), jnp.float32),
            pltpu.VMEM_SHARED((_NP, 16), jnp.float32),
            pltpu.SemaphoreType.DMA,
        ],
    )
    return f(asrc, adst, ctbl, srcp, dstp)


def _make_sc1(p):
    def body(hcat, asrc, adst, ctbl, srcp, dstp, accm_out,
             asrc_v, adst_v, ctbl_v, src_b, dst_b, src2_b, dsc_b,
             rows0, rows1, wtmp_v, accm_s, gsem0, gsem1):
        c = lax.axis_index("c")
        s = lax.axis_index("s")
        cP = c * _NP
        hoff = (c * 2 + p) * _N
        myhdK = c * _K

        pltpu.sync_copy(asrc, asrc_v)
        pltpu.sync_copy(adst, adst_v)
        pltpu.sync_copy(ctbl, ctbl_v)
        _zero_acc(rows0, None, accm_s, None, s, 4)

        tile_base = s * _ET1

        def _mul_rows(rows_v):
            def _edge(k4, _):
                k = k4 * 4
                kv = jnp.full((16,), k, jnp.int32) + myhdK
                for u in range(4):
                    sp = plsc.load_gather(wtmp_v, [kv + u])
                    for v in range(4):
                        rows_v[k + u, pl.ds(v * 16, 16)] = (
                            rows_v[k + u, pl.ds(v * 16, 16)] * sp)
                return 0
            lax.fori_loop(0, _K // 4, _edge, 0)

        def _drain_rows(rows_v, sem):
            pltpu.make_async_copy(accm_out.at[pl.ds(0, _K), :], rows_v, sem).wait()

        def _pipe(j, rowsA, gsemA, rowsB, gsemB):
            base = tile_base + j * _K
            pltpu.sync_copy(srcp.at[tile_row + j + 1], src_b)
            pltpu.sync_copy(dstp.at[tile_row + j + 1], dst_b)
            _drain_rows(rowsA, gsemA)
            _mul_rows(rowsA)
            pltpu.sync_copy(rowsA, accm_s.at[dsc_b], add=True)
            for g in range(8):
                src2_b[pl.ds(g * 16, 16)] = src_b[pl.ds(g * 16, 16)] + hoff
            pltpu.async_copy(hcat.at[src2_b], rowsB, gsemB)
            _edge_w2(asrc_v, adst_v, ctbl_v, src_b, dst_b, wtmp_v, base + _K)
            for g in range(8):
                dsc_b[pl.ds(g * 16, 16)] = dst_b[pl.ds(g * 16, 16)]

        tile_row = s * _NC1
        pltpu.sync_copy(srcp.at[tile_row], src_b)
        pltpu.sync_copy(dstp.at[tile_row], dst_b)
        for g in range(8):
            src2_b[pl.ds(g * 16, 16)] = src_b[pl.ds(g * 16, 16)] + hoff
        pltpu.async_copy(hcat.at[src2_b], rows0, gsem0)
        _edge_w2(asrc_v, adst_v, ctbl_v, src_b, dst_b, wtmp_v, tile_base)
        for g in range(8):
            dsc_b[pl.ds(g * 16, 16)] = dst_b[pl.ds(g * 16, 16)]

        def _pair(g2, _):
            j0 = g2 * 2
            _pipe(j0, rows0, gsem0, rows1, gsem1)
            _pipe(j0 + 1, rows1, gsem1, rows0, gsem0)
            return 0
        lax.fori_loop(0, _NC1 // 2, _pair, 0)
        _drain_rows(rows0, gsem0)

        plsc.subcore_barrier()
        _write_acc(accm_s, rows0, accm_out, s, cP)

    return pl.kernel(
        body, mesh=plsc.VectorSubcoreMesh(**_MESH),
        compiler_params=_SC_PARAMS,
        out_type=[jax.ShapeDtypeStruct((2 * _NP, 64), jnp.float32)],
        scratch_types=[
            pltpu.VMEM((_N * 2,), jnp.float32),
            pltpu.VMEM((_N * 2,), jnp.float32),
            pltpu.VMEM((32,), jnp.float32),
            pltpu.VMEM((_K,), jnp.int32),
            pltpu.VMEM((_K,), jnp.int32),
            pltpu.VMEM((_K,), jnp.int32),
            pltpu.VMEM((_K,), jnp.int32),
            pltpu.VMEM((_K, 64), jnp.float32),
            pltpu.VMEM((_K, 64), jnp.float32),
            pltpu.VMEM((2 * _K,), jnp.float32),
            pltpu.VMEM_SHARED((_NP, 64), jnp.float32),
            pltpu.SemaphoreType.DMA,
            pltpu.SemaphoreType.DMA,
        ],
    )


def _sc2_body(h2, asrc2, adst2, ctbl2, srcp, dstp, accm_out, accd_out,
              asrc_v, adst_v, ctbl_v, src_b, dst_b, src2_b, dsc_b,
              rows0, rows1, pad0, pad1, wtmp_v, accm_s, accd_s, gsem0, gsem1):
    c = lax.axis_index("c")
    s = lax.axis_index("s")
    cP = c * _NP
    cN = c * _N
    pltpu.sync_copy(asrc2, asrc_v)
    pltpu.sync_copy(adst2, adst_v)
    pltpu.sync_copy(ctbl2, ctbl_v)
    _zero_acc(rows0, pad0, accm_s, accd_s, s, 2)

    tile_base = s * _ET1
    tile_row = s * _NC1
    lanes = _iota16()

    def _w2(base):
        for g in range(8):
            sv = src_b[pl.ds(g * 16, 16)]
            dv = dst_b[pl.ds(g * 16, 16)]
            eid = base + g * 16 + _iota16()
            e = plsc.load_gather(asrc_v, [sv]) + plsc.load_gather(adst_v, [dv])
            e = jnp.maximum(e, 0.2 * e) - ctbl_v[pl.ds(0, 16)]
            w = jnp.where(eid < _E, jnp.exp(e), 0.0)
            wtmp_v[pl.ds(g * 16, 16)] = w

    def _pad_build(pad_v):
        def _edge(k4, _):
            k = k4 * 4
            kv = jnp.full((16,), k, jnp.int32)
            for u in range(4):
                sp = plsc.load_gather(wtmp_v, [kv + u])
                pad_v[k + u, :] = jnp.where(lanes < 1, sp, 0.0)
            return 0
        lax.fori_loop(0, _K // 4, _edge, 0)

    def _mul2(rows_v):
        def _edge(k4, _):
            k = k4 * 4
            kv = jnp.full((16,), k, jnp.int32)
            for u in range(4):
                sp = plsc.load_gather(wtmp_v, [kv + u])
                for v in range(2):
                    rows_v[k + u, pl.ds(v * 16, 16)] = (
                        rows_v[k + u, pl.ds(v * 16, 16)] * sp)
            return 0
        lax.fori_loop(0, _K // 4, _edge, 0)

    def _drain(rows_v, sem):
        pltpu.make_async_copy(h2.at[pl.ds(0, _K), :], rows_v, sem).wait()

    def _pipe(j, rowsA, gsemA, padA, rowsB, gsemB, padB):
        pltpu.sync_copy(srcp.at[tile_row + j + 1], src_b)
        pltpu.sync_copy(dstp.at[tile_row + j + 1], dst_b)
        _drain(rowsA, gsemA)
        _mul2(rowsA)
        pltpu.sync_copy(rowsA, accm_s.at[dsc_b], add=True)
        pltpu.sync_copy(padA, accd_s.at[dsc_b], add=True)
        for g in range(8):
            src2_b[pl.ds(g * 16, 16)] = src_b[pl.ds(g * 16, 16)] + cN
        pltpu.async_copy(h2.at[src2_b], rowsB, gsemB)
        _w2(tile_base + (j + 1) * _K)
        _pad_build(padB)
        for g in range(8):
            dsc_b[pl.ds(g * 16, 16)] = dst_b[pl.ds(g * 16, 16)]

    pltpu.sync_copy(srcp.at[tile_row], src_b)
    pltpu.sync_copy(dstp.at[tile_row], dst_b)
    for g in range(8):
        src2_b[pl.ds(g * 16, 16)] = src_b[pl.ds(g * 16, 16)] + cN
    pltpu.async_copy(h2.at[src2_b], rows0, gsem0)
    _w2(tile_base)
    _pad_build(pad0)
    for g in range(8):
        dsc_b[pl.ds(g * 16, 16)] = dst_b[pl.ds(g * 16, 16)]

    def _pair(g2, _):
        j0 = g2 * 2
        _pipe(j0, rows0, gsem0, pad0, rows1, gsem1, pad1)
        _pipe(j0 + 1, rows1, gsem1, pad1, rows0, gsem0, pad0)
        return 0
    lax.fori_loop(0, _NC1 // 2, _pair, 0)
    _drain(rows0, gsem0)

    plsc.subcore_barrier()
    _write_acc(accm_s, rows0, accm_out, s, cP)
    _write_acc(accd_s, pad0, accd_out, s, cP)


def _sc2(h2, asrc2, adst2, ctbl2, srcp, dstp):
    f = pl.kernel(
        _sc2_body, mesh=plsc.VectorSubcoreMesh(**_MESH),
        compiler_params=_SC_PARAMS,
        out_type=[
            jax.ShapeDtypeStruct((2 * _NP, 32), jnp.float32),
            jax.ShapeDtypeStruct((2 * _NP, 16), jnp.float32),
        ],
        scratch_types=[
            pltpu.VMEM((_N,), jnp.float32),
            pltpu.VMEM((_N,), jnp.float32),
            pltpu.VMEM((16,), jnp.float32),
            pltpu.VMEM((_K,), jnp.int32),
            pltpu.VMEM((_K,), jnp.int32),
            pltpu.VMEM((_K,), jnp.int32),
            pltpu.VMEM((_K,), jnp.int32),
            pltpu.VMEM((_K, 32), jnp.float32),
            pltpu.VMEM((_K, 32), jnp.float32),
            pltpu.VMEM((_K, 16), jnp.float32),
            pltpu.VMEM((_K, 16), jnp.float32),
            pltpu.VMEM((_K,), jnp.float32),
            pltpu.VMEM_SHARED((_NP, 32), jnp.float32),
            pltpu.VMEM_SHARED((_NP, 16), jnp.float32),
            pltpu.SemaphoreType.DMA,
            pltpu.SemaphoreType.DMA,
        ],
    )
    return f(h2, asrc2, adst2, ctbl2, srcp, dstp)


def kernel(x, edge_index, W1, a_src1, a_dst1, b1, W2, a_src2, a_dst2, b2,
           W_fc, b_fc):
    a_s = jnp.zeros((256, _HEADS), jnp.float32)
    a_d = jnp.zeros((256, _HEADS), jnp.float32)
    for hd in range(_HEADS):
        a_s = a_s.at[hd * 64:(hd + 1) * 64, hd].set(a_src1[hd])
        a_d = a_d.at[hd * 64:(hd + 1) * 64, hd].set(a_dst1[hd])

    src = edge_index[0]
    dst = edge_index[1]
    npad = _EP + 2 * _K - _E
    srcp = jnp.concatenate([src, jnp.zeros((npad,), jnp.int32)]).reshape(-1, _K)
    dstp = jnp.concatenate(
        [dst, (jnp.arange(npad, dtype=jnp.int32) * 97) % _N]).reshape(-1, _K)

    hcat, asrc, adst, ctbl, root = _tc1(x, W1, a_s, a_d)
    asrc_f, adst_f, ctbl_f = asrc.reshape(-1), adst.reshape(-1), ctbl.reshape(-1)
    (accd1,) = _sc_den1(asrc_f, adst_f, ctbl_f, srcp, dstp)
    asrc_t, adst_t = asrc.T, adst.T
    accm_p = []
    for pp in range(2):
        a_p = jnp.concatenate([asrc_t[pp], asrc_t[pp + 2]])
        d_p = jnp.concatenate([adst_t[pp], adst_t[pp + 2]])
        c_p = jnp.concatenate([ctbl[pp], ctbl[pp + 2]])
        accm_p.append(_make_sc1(pp)(hcat, a_p, d_p, c_p, srcp, dstp)[0])
    accm_p0, accm_p1 = accm_p
    h2, asrc2, adst2 = _tc2(
        accm_p0[0:_N], accm_p1[0:_N], accm_p0[_NP:_NP + _N],
        accm_p1[_NP:_NP + _N], accd1[0:_N], accd1[_NP:_NP + _N],
        b1, W2, a_src2, a_dst2)
    ctbl2 = _tcc2(asrc2, adst2)
    h2cat = jnp.concatenate([h2[:, 0:32], h2[:, 32:64]], axis=0)
    accm2, accd2 = _sc2(h2cat, asrc2.reshape(-1), adst2.reshape(-1),
                        ctbl2.reshape(-1), srcp, dstp)
    out = _tc3(accm2[0:_N], accm2[_NP:_NP + _N], accd2[0:_N], b2, W_fc, b_fc)
    return out[root[0, 0]][None, :]


# speedup vs baseline: 28.4047x; 1.1392x over previous
import jax
import jax.numpy as jnp
from jax import lax
from jax.experimental import pallas as pl
from jax.experimental.pallas import tpu as pltpu
from jax.experimental.pallas import tpu_sc as plsc

_N = 10000
_E = 320000
_HEADS = 4

_K = 128
_NTILES = 16
_NCORES = 2

_EP = _NCORES * _NTILES * _K * ((_E + _NCORES * _NTILES * _K - 1) // (_NCORES * _NTILES * _K))
_ET1 = _EP // _NTILES
_NC1 = _ET1 // _K
_ET2 = _EP // (_NCORES * _NTILES)
_NC2 = _ET2 // _K

_NP = 10240
_ROWS_PER_TILE = _NP // _NTILES


def _iota16():
    return lax.iota(jnp.int32, 16)


def _tc1_body(x_ref, w1_ref, as_ref, ad_ref, hcat_ref, asrc_ref, adst_ref,
              ctbl_ref, root_ref):
    x = x_ref[...]
    h = jnp.dot(x, w1_ref[...], preferred_element_type=jnp.float32)
    for hd in range(4):
        hcat_ref[hd * _N:(hd + 1) * _N, :] = h[:, hd * 64:(hd + 1) * 64]
    asrc = jnp.dot(h, as_ref[...], preferred_element_type=jnp.float32)
    adst = jnp.dot(h, ad_ref[...], preferred_element_type=jnp.float32)
    asrc_ref[...] = asrc
    adst_ref[...] = adst
    c = jnp.maximum(jnp.max(asrc, axis=0) + jnp.max(adst, axis=0), 0.0)
    ctbl_ref[...] = jnp.broadcast_to(c[:, None], (4, 16))
    mask = x[:, 0:1] == 0.0
    ids = lax.broadcasted_iota(jnp.int32, (_N, 1), 0)
    rid = jnp.min(jnp.where(mask, ids, _N))
    root_ref[...] = jnp.broadcast_to(jnp.where(rid == _N, 0, rid), (1, 1))


def _tc1(x, w1, a_s, a_d):
    return pl.pallas_call(
        _tc1_body,
        out_shape=[
            jax.ShapeDtypeStruct((4 * _N, 64), jnp.float32),
            jax.ShapeDtypeStruct((_N, 4), jnp.float32),
            jax.ShapeDtypeStruct((_N, 4), jnp.float32),
            jax.ShapeDtypeStruct((4, 16), jnp.float32),
            jax.ShapeDtypeStruct((1, 1), jnp.int32),
        ],
    )(x, w1, a_s, a_d)


_B2 = 2000


def _tc2_body(a0l_ref, a1l_ref, a0h_ref, a1h_ref, adl_ref, adh_ref,
              b1_ref, w2_ref, as2_ref, ad2_ref,
              h2_ref, asrc2_ref, adst2_ref):
    den = adl_ref[:, 0:4] + adh_ref[:, 0:4] + 1e-16
    num = jnp.concatenate([a0l_ref[...], a1l_ref[...],
                           a0h_ref[...], a1h_ref[...]], axis=1)
    scale = jnp.concatenate(
        [jnp.broadcast_to(1.0 / den[:, hd:hd + 1], (_B2, 64)) for hd in range(4)],
        axis=1)
    h1 = jnp.maximum(num * scale + b1_ref[...][None, :], 0.0)
    h2 = jnp.dot(h1, w2_ref[...], preferred_element_type=jnp.float32)
    h2_ref[...] = h2
    asrc2_ref[...] = jnp.sum(h2 * as2_ref[...], axis=1, keepdims=True)
    adst2_ref[...] = jnp.sum(h2 * ad2_ref[...], axis=1, keepdims=True)


def _tc2(a0l, a1l, a0h, a1h, adl, adh, b1, w2, a_s2, a_d2):
    nblk = _N // _B2
    row = lambda i: (i, 0)
    full2 = lambda i: (0, 0)
    return pl.pallas_call(
        _tc2_body,
        grid=(nblk,),
        in_specs=[
            pl.BlockSpec((_B2, 64), row), pl.BlockSpec((_B2, 64), row),
            pl.BlockSpec((_B2, 64), row), pl.BlockSpec((_B2, 64), row),
            pl.BlockSpec((_B2, 16), row), pl.BlockSpec((_B2, 16), row),
            pl.BlockSpec((256,), lambda i: (0,)),
            pl.BlockSpec((256, 64), full2),
            pl.BlockSpec((1, 64), full2), pl.BlockSpec((1, 64), full2),
        ],
        out_specs=[
            pl.BlockSpec((_B2, 64), row),
            pl.BlockSpec((_B2, 1), row),
            pl.BlockSpec((_B2, 1), row),
        ],
        out_shape=[
            jax.ShapeDtypeStruct((_N, 64), jnp.float32),
            jax.ShapeDtypeStruct((_N, 1), jnp.float32),
            jax.ShapeDtypeStruct((_N, 1), jnp.float32),
        ],
    )(a0l, a1l, a0h, a1h, adl, adh, b1, w2, a_s2, a_d2)


def _tcc2_body(asrc2_ref, adst2_ref, ctbl2_ref):
    c2 = jnp.maximum(jnp.max(asrc2_ref[...]) + jnp.max(adst2_ref[...]), 0.0)
    ctbl2_ref[...] = jnp.full((1, 16), 1.0) * c2


def _tcc2(asrc2, adst2):
    return pl.pallas_call(
        _tcc2_body,
        out_shape=jax.ShapeDtypeStruct((1, 16), jnp.float32),
    )(asrc2, adst2)


def _tc3_body(m2l_ref, m2h_ref, d2l_ref, b2_ref, wfc_ref, bfc_ref, out_ref):
    num = jnp.concatenate([m2l_ref[...], m2h_ref[...]], axis=1)
    den = d2l_ref[:, 0:1] + 1e-16
    h2 = jnp.maximum(num / den + b2_ref[...][None, :], 0.0)
    out_ref[...] = (jnp.dot(h2, wfc_ref[...], preferred_element_type=jnp.float32)
                    + bfc_ref[...][None, :])


def _tc3(m2l, m2h, d2l, b2, wfc, bfc):
    return pl.pallas_call(
        _tc3_body,
        out_shape=jax.ShapeDtypeStruct((_N, 64), jnp.float32),
    )(m2l, m2h, d2l, b2, wfc, bfc)


_SC_PARAMS = pltpu.CompilerParams(needs_layout_passes=False,
                                  use_tc_tiling_on_sc=False)
_MESH = dict(core_axis_name="c", subcore_axis_name="s")


def _zero_acc(rows_v, pad_v, accm_s, accd_s, s, nvec):
    def _zrow(i, _):
        if rows_v is not None:
            for v in range(nvec):
                rows_v[i, pl.ds(v * 16, 16)] = jnp.zeros((16,), jnp.float32)
        if pad_v is not None:
            pad_v[i, :] = jnp.zeros((16,), jnp.float32)
        return 0
    lax.fori_loop(0, _K, _zrow, 0)
    for r in range(0, _ROWS_PER_TILE, _K):
        if accm_s is not None:
            pltpu.sync_copy(rows_v, accm_s.at[pl.ds(s * _ROWS_PER_TILE + r, _K), :])
        if accd_s is not None:
            pltpu.sync_copy(pad_v, accd_s.at[pl.ds(s * _ROWS_PER_TILE + r, _K), :])
    plsc.subcore_barrier()


def _write_acc(acc_s, buf_v, out_hbm, s, cP):
    for r in range(0, _ROWS_PER_TILE, _K):
        row0 = s * _ROWS_PER_TILE + r
        pltpu.sync_copy(acc_s.at[pl.ds(row0, _K), :], buf_v)
        pltpu.sync_copy(buf_v, out_hbm.at[pl.ds(cP + row0, _K), :])


def _edge_w(asrc_v, adst_v, ctbl_v, src_b, dst_b, wtmp_v, base, heads, nh):
    for g in range(8):
        sv = src_b[pl.ds(g * 16, 16)] * nh
        dv = dst_b[pl.ds(g * 16, 16)] * nh
        eid = base + g * 16 + _iota16()
        valid = eid < _E
        for hd in heads:
            e = (plsc.load_gather(asrc_v, [sv + hd])
                 + plsc.load_gather(adst_v, [dv + hd]))
            e = jnp.maximum(e, 0.2 * e) - ctbl_v[pl.ds(hd * 16, 16)]
            w = jnp.where(valid, jnp.exp(e), 0.0)
            wtmp_v[pl.ds(hd * _K + g * 16, 16)] = w


def _edge_w2(asrc_v, adst_v, ctbl_v, src_b, dst_b, wtmp_v, base):
    for g in range(8):
        sv = src_b[pl.ds(g * 16, 16)]
        dv = dst_b[pl.ds(g * 16, 16)]
        eid = base + g * 16 + _iota16()
        valid = eid < _E
        for slot in range(2):
            e = (plsc.load_gather(asrc_v, [sv + slot * _N])
                 + plsc.load_gather(adst_v, [dv + slot * _N]))
            e = jnp.maximum(e, 0.2 * e) - ctbl_v[pl.ds(slot * 16, 16)]
            w = jnp.where(valid, jnp.exp(e), 0.0)
            wtmp_v[pl.ds(slot * _K + g * 16, 16)] = w


def _sc_den1_body(asrc, adst, ctbl, srcp, dstp, accd_out,
                  asrc_v, adst_v, ctbl_v, src_b, dst_b, pad_v, wtmp_v, accd_s,
                  sem):
    c = lax.axis_index("c")
    s = lax.axis_index("s")
    cP = c * _NP
    pltpu.sync_copy(asrc, asrc_v)
    pltpu.sync_copy(adst, adst_v)
    pltpu.sync_copy(ctbl, ctbl_v)
    _zero_acc(None, pad_v, None, accd_s, s, 0)

    tile_base = (c * _NTILES + s) * _ET2
    lanes = _iota16()

    tile_row = (c * _NTILES + s) * _NC2

    def _chunk(j, _):
        base = tile_base + j * _K
        pltpu.sync_copy(srcp.at[tile_row + j], src_b)
        pltpu.sync_copy(dstp.at[tile_row + j], dst_b)
        _edge_w(asrc_v, adst_v, ctbl_v, src_b, dst_b, wtmp_v, base,
                (0, 1, 2, 3), 4)

        def _edge(k4, _):
            k = k4 * 4
            kv = (lanes & 3) * _K + jnp.full((16,), k, jnp.int32)
            for u in range(4):
                padv = plsc.load_gather(wtmp_v, [kv + u])
                pad_v[k + u, :] = jnp.where(lanes < 4, padv, 0.0)
            return 0
        lax.fori_loop(0, _K // 4, _edge, 0)

        pltpu.sync_copy(pad_v, accd_s.at[dst_b], add=True)
        return 0

    lax.fori_loop(0, _NC2, _chunk, 0)
    plsc.subcore_barrier()
    _write_acc(accd_s, pad_v, accd_out, s, cP)


def _sc_den1(asrc, adst, ctbl, srcp, dstp):
    f = pl.kernel(
        _sc_den1_body, mesh=plsc.VectorSubcoreMesh(**_MESH),
        compiler_params=_SC_PARAMS,
        out_type=[jax.ShapeDtypeStruct((2 * _NP, 16), jnp.float32)],
        scratch_types=[
            pltpu.VMEM((_N * 4,), jnp.float32),
            pltpu.VMEM((_N * 4,), jnp.float32),
            pltpu.VMEM((64,), jnp.float32),
            pltpu.VMEM((_K,), jnp.int32),
            pltpu.VMEM((_K,), jnp.int32),
            pltpu.VMEM((_K, 16), jnp.float32),
            pltpu.VMEM((4 * _K,), jnp.float32),
            pltpu.VMEM_SHARED((_NP, 16), jnp.float32),
            pltpu.SemaphoreType.DMA,
        ],
    )
    return f(asrc, adst, ctbl, srcp, dstp)


def _make_sc1(p):
    def body(hcat, asrc, adst, ctbl, srcp, dstp, accm_out,
             asrc_v, adst_v, ctbl_v, src_b, dst_b, src2_b, dsc_b,
             rows0, rows1, wtmp_v, accm_s, gsem0, gsem1):
        c = lax.axis_index("c")
        s = lax.axis_index("s")
        cP = c * _NP
        hoff = (c * 2 + p) * _N
        myhdK = c * _K

        pltpu.sync_copy(asrc, asrc_v)
        pltpu.sync_copy(adst, adst_v)
        pltpu.sync_copy(ctbl, ctbl_v)
        _zero_acc(rows0, None, accm_s, None, s, 4)

        tile_base = s * _ET1

        def _mul_rows(rows_v):
            def _edge(k16, _):
                k = k16 * 16
                w16 = wtmp_v[pl.ds(k + myhdK, 16)]
                for u in range(16):
                    sp = jnp.full((16,), w16[u], jnp.float32)
                    for v in range(4):
                        rows_v[k + u, pl.ds(v * 16, 16)] = (
                            rows_v[k + u, pl.ds(v * 16, 16)] * sp)
                return 0
            lax.fori_loop(0, _K // 16, _edge, 0)

        def _drain_rows(rows_v, sem):
            pltpu.make_async_copy(accm_out.at[pl.ds(0, _K), :], rows_v, sem).wait()

        def _pipe(j, rowsA, gsemA, rowsB, gsemB):
            base = tile_base + j * _K
            pltpu.sync_copy(srcp.at[tile_row + j + 1], src_b)
            pltpu.sync_copy(dstp.at[tile_row + j + 1], dst_b)
            _drain_rows(rowsA, gsemA)
            _mul_rows(rowsA)
            pltpu.sync_copy(rowsA, accm_s.at[dsc_b], add=True)
            for g in range(8):
                src2_b[pl.ds(g * 16, 16)] = src_b[pl.ds(g * 16, 16)] + hoff
            pltpu.async_copy(hcat.at[src2_b], rowsB, gsemB)
            _edge_w2(asrc_v, adst_v, ctbl_v, src_b, dst_b, wtmp_v, base + _K)
            for g in range(8):
                dsc_b[pl.ds(g * 16, 16)] = dst_b[pl.ds(g * 16, 16)]

        tile_row = s * _NC1
        pltpu.sync_copy(srcp.at[tile_row], src_b)
        pltpu.sync_copy(dstp.at[tile_row], dst_b)
        for g in range(8):
            src2_b[pl.ds(g * 16, 16)] = src_b[pl.ds(g * 16, 16)] + hoff
        pltpu.async_copy(hcat.at[src2_b], rows0, gsem0)
        _edge_w2(asrc_v, adst_v, ctbl_v, src_b, dst_b, wtmp_v, tile_base)
        for g in range(8):
            dsc_b[pl.ds(g * 16, 16)] = dst_b[pl.ds(g * 16, 16)]

        def _pair(g2, _):
            j0 = g2 * 2
            _pipe(j0, rows0, gsem0, rows1, gsem1)
            _pipe(j0 + 1, rows1, gsem1, rows0, gsem0)
            return 0
        lax.fori_loop(0, _NC1 // 2, _pair, 0)
        _drain_rows(rows0, gsem0)

        plsc.subcore_barrier()
        _write_acc(accm_s, rows0, accm_out, s, cP)

    return pl.kernel(
        body, mesh=plsc.VectorSubcoreMesh(**_MESH),
        compiler_params=_SC_PARAMS,
        out_type=[jax.ShapeDtypeStruct((2 * _NP, 64), jnp.float32)],
        scratch_types=[
            pltpu.VMEM((_N * 2,), jnp.float32),
            pltpu.VMEM((_N * 2,), jnp.float32),
            pltpu.VMEM((32,), jnp.float32),
            pltpu.VMEM((_K,), jnp.int32),
            pltpu.VMEM((_K,), jnp.int32),
            pltpu.VMEM((_K,), jnp.int32),
            pltpu.VMEM((_K,), jnp.int32),
            pltpu.VMEM((_K, 64), jnp.float32),
            pltpu.VMEM((_K, 64), jnp.float32),
            pltpu.VMEM((2 * _K,), jnp.float32),
            pltpu.VMEM_SHARED((_NP, 64), jnp.float32),
            pltpu.SemaphoreType.DMA,
            pltpu.SemaphoreType.DMA,
        ],
    )


def _sc2_body(h2, asrc2, adst2, ctbl2, srcp, dstp, accm_out, accd_out,
              asrc_v, adst_v, ctbl_v, src_b, dst_b, src2_b, dsc_b,
              rows0, rows1, pad0, pad1, wtmp_v, accm_s, accd_s, gsem0, gsem1):
    c = lax.axis_index("c")
    s = lax.axis_index("s")
    cP = c * _NP
    cN = c * _N
    pltpu.sync_copy(asrc2, asrc_v)
    pltpu.sync_copy(adst2, adst_v)
    pltpu.sync_copy(ctbl2, ctbl_v)
    _zero_acc(rows0, pad0, accm_s, accd_s, s, 2)

    tile_base = s * _ET1
    tile_row = s * _NC1
    lanes = _iota16()

    def _w2(base):
        for g in range(8):
            sv = src_b[pl.ds(g * 16, 16)]
            dv = dst_b[pl.ds(g * 16, 16)]
            eid = base + g * 16 + _iota16()
            e = plsc.load_gather(asrc_v, [sv]) + plsc.load_gather(adst_v, [dv])
            e = jnp.maximum(e, 0.2 * e) - ctbl_v[pl.ds(0, 16)]
            w = jnp.where(eid < _E, jnp.exp(e), 0.0)
            wtmp_v[pl.ds(g * 16, 16)] = w

    def _pad_build(pad_v):
        def _edge(k16, _):
            k = k16 * 16
            w16 = wtmp_v[pl.ds(k, 16)]
            for u in range(16):
                pad_v[k + u, :] = jnp.where(lanes < 1,
                                            jnp.full((16,), w16[u], jnp.float32),
                                            0.0)
            return 0
        lax.fori_loop(0, _K // 16, _edge, 0)

    def _mul2(rows_v):
        def _edge(k16, _):
            k = k16 * 16
            w16 = wtmp_v[pl.ds(k, 16)]
            for u in range(16):
                sp = jnp.full((16,), w16[u], jnp.float32)
                for v in range(2):
                    rows_v[k + u, pl.ds(v * 16, 16)] = (
                        rows_v[k + u, pl.ds(v * 16, 16)] * sp)
            return 0
        lax.fori_loop(0, _K // 16, _edge, 0)

    def _drain(rows_v, sem):
        pltpu.make_async_copy(h2.at[pl.ds(0, _K), :], rows_v, sem).wait()

    def _pipe(j, rowsA, gsemA, padA, rowsB, gsemB, padB):
        pltpu.sync_copy(srcp.at[tile_row + j + 1], src_b)
        pltpu.sync_copy(dstp.at[tile_row + j + 1], dst_b)
        _drain(rowsA, gsemA)
        _mul2(rowsA)
        pltpu.sync_copy(rowsA, accm_s.at[dsc_b], add=True)
        pltpu.sync_copy(padA, accd_s.at[dsc_b], add=True)
        for g in range(8):
            src2_b[pl.ds(g * 16, 16)] = src_b[pl.ds(g * 16, 16)] + cN
        pltpu.async_copy(h2.at[src2_b], rowsB, gsemB)
        _w2(tile_base + (j + 1) * _K)
        _pad_build(padB)
        for g in range(8):
            dsc_b[pl.ds(g * 16, 16)] = dst_b[pl.ds(g * 16, 16)]

    pltpu.sync_copy(srcp.at[tile_row], src_b)
    pltpu.sync_copy(dstp.at[tile_row], dst_b)
    for g in range(8):
        src2_b[pl.ds(g * 16, 16)] = src_b[pl.ds(g * 16, 16)] + cN
    pltpu.async_copy(h2.at[src2_b], rows0, gsem0)
    _w2(tile_base)
    _pad_build(pad0)
    for g in range(8):
        dsc_b[pl.ds(g * 16, 16)] = dst_b[pl.ds(g * 16, 16)]

    def _pair(g2, _):
        j0 = g2 * 2
        _pipe(j0, rows0, gsem0, pad0, rows1, gsem1, pad1)
        _pipe(j0 + 1, rows1, gsem1, pad1, rows0, gsem0, pad0)
        return 0
    lax.fori_loop(0, _NC1 // 2, _pair, 0)
    _drain(rows0, gsem0)

    plsc.subcore_barrier()
    _write_acc(accm_s, rows0, accm_out, s, cP)
    _write_acc(accd_s, pad0, accd_out, s, cP)


def _sc2(h2, asrc2, adst2, ctbl2, srcp, dstp):
    f = pl.kernel(
        _sc2_body, mesh=plsc.VectorSubcoreMesh(**_MESH),
        compiler_params=_SC_PARAMS,
        out_type=[
            jax.ShapeDtypeStruct((2 * _NP, 32), jnp.float32),
            jax.ShapeDtypeStruct((2 * _NP, 16), jnp.float32),
        ],
        scratch_types=[
            pltpu.VMEM((_N,), jnp.float32),
            pltpu.VMEM((_N,), jnp.float32),
            pltpu.VMEM((16,), jnp.float32),
            pltpu.VMEM((_K,), jnp.int32),
            pltpu.VMEM((_K,), jnp.int32),
            pltpu.VMEM((_K,), jnp.int32),
            pltpu.VMEM((_K,), jnp.int32),
            pltpu.VMEM((_K, 32), jnp.float32),
            pltpu.VMEM((_K, 32), jnp.float32),
            pltpu.VMEM((_K, 16), jnp.float32),
            pltpu.VMEM((_K, 16), jnp.float32),
            pltpu.VMEM((_K,), jnp.float32),
            pltpu.VMEM_SHARED((_NP, 32), jnp.float32),
            pltpu.VMEM_SHARED((_NP, 16), jnp.float32),
            pltpu.SemaphoreType.DMA,
            pltpu.SemaphoreType.DMA,
        ],
    )
    return f(h2, asrc2, adst2, ctbl2, srcp, dstp)


def kernel(x, edge_index, W1, a_src1, a_dst1, b1, W2, a_src2, a_dst2, b2,
           W_fc, b_fc):
    a_s = jnp.zeros((256, _HEADS), jnp.float32)
    a_d = jnp.zeros((256, _HEADS), jnp.float32)
    for hd in range(_HEADS):
        a_s = a_s.at[hd * 64:(hd + 1) * 64, hd].set(a_src1[hd])
        a_d = a_d.at[hd * 64:(hd + 1) * 64, hd].set(a_dst1[hd])

    src = edge_index[0]
    dst = edge_index[1]
    npad = _EP + 2 * _K - _E
    srcp = jnp.concatenate([src, jnp.zeros((npad,), jnp.int32)]).reshape(-1, _K)
    dstp = jnp.concatenate(
        [dst, (jnp.arange(npad, dtype=jnp.int32) * 97) % _N]).reshape(-1, _K)

    hcat, asrc, adst, ctbl, root = _tc1(x, W1, a_s, a_d)
    asrc_f, adst_f, ctbl_f = asrc.reshape(-1), adst.reshape(-1), ctbl.reshape(-1)
    (accd1,) = _sc_den1(asrc_f, adst_f, ctbl_f, srcp, dstp)
    asrc_t, adst_t = asrc.T, adst.T
    accm_p = []
    for pp in range(2):
        a_p = jnp.concatenate([asrc_t[pp], asrc_t[pp + 2]])
        d_p = jnp.concatenate([adst_t[pp], adst_t[pp + 2]])
        c_p = jnp.concatenate([ctbl[pp], ctbl[pp + 2]])
        accm_p.append(_make_sc1(pp)(hcat, a_p, d_p, c_p, srcp, dstp)[0])
    accm_p0, accm_p1 = accm_p
    h2, asrc2, adst2 = _tc2(
        accm_p0[0:_N], accm_p1[0:_N], accm_p0[_NP:_NP + _N],
        accm_p1[_NP:_NP + _N], accd1[0:_N], accd1[_NP:_NP + _N],
        b1, W2, a_src2, a_dst2)
    ctbl2 = _tcc2(asrc2, adst2)
    h2cat = jnp.concatenate([h2[:, 0:32], h2[:, 32:64]], axis=0)
    accm2, accd2 = _sc2(h2cat, asrc2.reshape(-1), adst2.reshape(-1),
                        ctbl2.reshape(-1), srcp, dstp)
    out = _tc3(accm2[0:_N], accm2[_NP:_NP + _N], accd2[0:_N], b2, W_fc, b_fc)
    return out[root[0, 0]][None, :]
